# baseline (speedup 1.0000x reference)
import functools

import jax
import jax.numpy as jnp
from jax import lax
from jax.experimental import pallas as pl
from jax.experimental.pallas import tpu as pltpu

_ML = 2.9086
_SL = 1.898
_SCALE_MR = 500.0
_INV_SCALE_MR = 1.0 / _SCALE_MR
_LANES = 128

_COL_H = 0
_COL_C = 1
_COL_L = 2
_COL_LC = 3
_COL_BP = 4
_COL_IB = 5
_COL_OO = 6
_COL_OL = 7
_COL_OLC = 8
_COL_F = 9
_COL_STD = 10
_COL_OV = 11

(_P_HOO1, _P_G1, _P_KOOH, _P_AOOH, _P_SIG, _P_EXP, _P_THR, _P_STD) = range(8)
_N_PARAMS = 8

_GROUP = 128


def _round_up(x, m):
    return (x + m - 1) // m * m


def _extract_kernel(x_ref, p2_ref, u1_ref, u2_ref, ol_ref):
    rows = x_ref.shape[0]
    ol1 = jnp.full((1, _LANES), p2_ref[0], dtype=jnp.float32)
    k_ol = jnp.full((1, _LANES), p2_ref[1], dtype=jnp.float32)
    a_ol = jnp.full((1, _LANES), p2_ref[2], dtype=jnp.float32)
    for g in range(rows // _LANES):
        t = x_ref[pl.ds(g * _LANES, _LANES), 0:8].T
        u2r = t[1:2, :]
        u1_ref[pl.ds(g, 1), :] = t[0:1, :]
        u2_ref[pl.ds(g, 1), :] = u2r
        ol_ref[pl.ds(g, 1), :] = ol1 * jax.nn.sigmoid(k_ol + u2r * a_ol)


def _rnn_kernel(u1_ref, u2_ref, ol_ref, p_ref, out_ref, c_state, scr_ref, *,
                batch, seq_len, time_lag):

    shape = (1, _LANES)

    def splat(j):
        return jnp.full(shape, p_ref[j], dtype=jnp.float32)

    hoo1 = splat(_P_HOO1)
    g1 = splat(_P_G1)
    koo_h = splat(_P_KOOH)
    aoo_h = splat(_P_AOOH)
    sig = splat(_P_SIG)
    exp_yrm = splat(_P_EXP)
    thr = splat(_P_THR)
    obs_std = splat(_P_STD)
    lane = lax.broadcasted_iota(jnp.int32, shape, 1)
    _used = (_COL_H, _COL_C, _COL_L, _COL_LC, _COL_OO, _COL_OL,
             _COL_OLC, _COL_F, _COL_STD, _COL_OV)
    onehot = {j: (lane == j).astype(jnp.float32) for j in _used}

    def step(c0pair, u1, u2, ol):
        c0a, c0b = c0pair
        c0 = jnp.maximum(c0a, c0b)
        cpos = c0 > 0.0
        ta = jnp.tanh(koo_h + c0a * aoo_h)
        tb = jnp.tanh(koo_h + c0b * aoo_h)
        t = jnp.where(c0a >= c0b, ta, tb)
        r = pl.reciprocal(c0, approx=True)
        d = c0 - thr
        ad = jnp.abs(d)
        cm = c0 - ad
        e = u1 - sig * d
        q = ol * c0
        olc_c0 = jnp.where(cpos, jnp.minimum(q, u2), q)
        olc = jnp.where(cpos, jnp.minimum(ol, u2 * r), ol)
        hc = hoo1 * c0
        hcm = hoo1 * cm
        k1 = (g1 * c0 + e) - olc_c0
        k2 = g1 * cm + u1
        c1a = k1 - t * hc
        c1b = (k2 - t * hcm) - olc * cm
        return (c1a, c1b), t, olc, q, olc_c0

    out_ref[:, pl.ds(0, time_lag)] = jnp.zeros((_LANES, time_lag),
                                               out_ref.dtype)

    def row_body(g, k, cp0):
        row_off = (g * _GROUP + k) * seq_len
        cp = cp0
        for t in range(seq_len - 1):
            cp = step(cp, u1_ref[row_off + t], u2_ref[row_off + t],
                      ol_ref[row_off + t])[0]
        idx = row_off + seq_len - 1
        ol = ol_ref[idx]
        c = jnp.maximum(cp[0], cp[1])
        cp_new, t, olc, q, olc_c0 = step(cp, u1_ref[idx], u2_ref[idx], ol)
        a1 = hoo1 * t
        oo = hoo1 + a1
        f = (g1 - a1) - olc
        sgn = jnp.sign(c * _INV_SCALE_MR - exp_yrm)
        ov = jnp.minimum(sig * sgn, f)
        packed = ((oo * c) * onehot[_COL_H]
                  + c * onehot[_COL_C]
                  + q * onehot[_COL_L]
                  + olc_c0 * onehot[_COL_LC]
                  + oo * onehot[_COL_OO]
                  + ol * onehot[_COL_OL]
                  + olc * onehot[_COL_OLC]
                  + f * onehot[_COL_F]
                  + obs_std * onehot[_COL_STD]
                  + ov * onehot[_COL_OV])
        scr_ref[pl.ds(k, 1), :] = packed
        return cp_new

    def group_body(g, cp0):
        cp1 = lax.fori_loop(0, _GROUP,
                            lambda k, cp: row_body(g, k, cp), cp0, unroll=2)
        out_ref[:, pl.ds(g * _GROUP, _GROUP)] = scr_ref[...].T
        return cp1

    zero = jnp.zeros(shape, jnp.float32)
    cp_final = lax.fori_loop(time_lag // _GROUP, batch // _GROUP,
                             group_body, (zero, zero))
    c_state[...] = jnp.maximum(cp_final[0], cp_final[1])


def _forward(x, y_obs, params, p_mean, p_std, *, time_lag, spin_len,
             train_len):
    batch, seq, _ = x.shape
    x = x.astype(jnp.float32)

    f32 = lambda v: jnp.asarray(v, jnp.float32).reshape(())
    w_r_yom = f32(params['weight_r_yom'])
    w_r_ylm = f32(params['weight_r_ylm'])
    w_r_yfm = f32(params['weight_r_yfm'])
    w_r_yvm = f32(params['weight_r_yvm'])
    b0_yom = f32(params['bias_b0_yom'])
    w_b1_yom = f32(params['weight_b1_yom'])
    b0_ylm = f32(params['bias_b0_ylm'])
    w_b2_ylm = f32(params['weight_b2_ylm'])
    b0_yrm = f32(params['bias_b0_yrm'])
    mo = f32(p_mean)
    so = f32(p_std)

    e_o, e_l, e_f = jnp.exp(w_r_yom), jnp.exp(w_r_ylm), jnp.exp(w_r_yfm)
    denom = e_o + e_l + e_f
    oo1 = e_o / denom
    ol1 = e_l / denom
    sig_yvm = jax.nn.sigmoid(w_r_yvm)
    exp_yrm = jnp.exp(b0_yrm)
    thr = exp_yrm * jnp.float32(_SCALE_MR)
    a_oo = w_b1_yom / so
    k_oo = b0_yom - mo * a_oo
    a_ol = w_b2_ylm / jnp.float32(_SL)
    k_ol = b0_ylm - jnp.float32(_ML) * a_ol
    obs_std = jnp.std(y_obs[spin_len:train_len].astype(jnp.float32), ddof=1)

    n = batch * seq
    x2 = x.reshape(n, _LANES)
    p2_vec = jnp.stack([ol1, k_ol, a_ol]).astype(jnp.float32)
    npar = 2
    nblk = max(1, n // (npar * 1024))
    rows_blk = n // (npar * nblk)
    u1_2d, u2_2d, ol_2d = pl.pallas_call(
        _extract_kernel,
        out_shape=[jax.ShapeDtypeStruct((n // _LANES, _LANES), jnp.float32)
                   for _ in range(3)],
        grid_spec=pltpu.PrefetchScalarGridSpec(
            num_scalar_prefetch=0,
            grid=(npar, nblk),
            in_specs=[
                pl.BlockSpec((rows_blk, _LANES),
                             lambda i, j: (i * nblk + j, 0)),
                pl.BlockSpec(memory_space=pltpu.MemorySpace.SMEM),
            ],
            out_specs=[
                pl.BlockSpec((rows_blk // _LANES, _LANES),
                             lambda i, j: (i * nblk + j, 0))
                for _ in range(3)
            ],
        ),
        compiler_params=pltpu.CompilerParams(
            dimension_semantics=("parallel", "arbitrary")),
    )(x2, p2_vec)
    u1 = u1_2d.reshape(-1)
    u2 = u2_2d.reshape(-1)
    ol_all = ol_2d.reshape(-1)

    hoo1 = 0.5 * oo1
    p_vec = jnp.stack([hoo1, 1.0 - hoo1, 0.5 * k_oo, 0.5 * a_oo, sig_yvm,
                       exp_yrm, thr, obs_std]).astype(jnp.float32)

    _kernel_fn = functools.partial(_rnn_kernel, batch=batch, seq_len=seq,
                                   time_lag=time_lag)

    out = pl.pallas_call(
        _kernel_fn,
        out_shape=jax.ShapeDtypeStruct((_LANES, batch), jnp.float32),
        grid_spec=pltpu.PrefetchScalarGridSpec(
            num_scalar_prefetch=0,
            grid=(1,),
            in_specs=[
                pl.BlockSpec(memory_space=pltpu.MemorySpace.SMEM),
                pl.BlockSpec(memory_space=pltpu.MemorySpace.SMEM),
                pl.BlockSpec(memory_space=pltpu.MemorySpace.SMEM),
                pl.BlockSpec(memory_space=pltpu.MemorySpace.SMEM),
            ],
            out_specs=pl.BlockSpec((_LANES, batch), lambda i: (0, 0)),
            scratch_shapes=[pltpu.VMEM((1, _LANES), jnp.float32),
                            pltpu.VMEM((_GROUP, _LANES), jnp.float32)],
        ),
        compiler_params=pltpu.CompilerParams(
            dimension_semantics=("arbitrary",)),
    )(u1, u2, ol_all, p_vec)

    col = lambda j: out[j].reshape(batch, 1)
    h_n = col(_COL_H)
    obs_std_col = col(_COL_STD)
    h_nout = jnp.concatenate([h_n, obs_std_col], axis=1)
    return (h_n, col(_COL_C), col(_COL_L), col(_COL_LC), col(_COL_BP),
            col(_COL_IB), col(_COL_OO), col(_COL_OL), col(_COL_OLC),
            col(_COL_F), h_nout, obs_std_col, col(_COL_OV))


def kernel(x, y_obs, weight_r_yom, weight_r_ylm, weight_r_yfm, weight_r_yvm,
           bias_b0_yom, weight_b1_yom, bias_b0_ylm, weight_b2_ylm,
           bias_b0_yrm, p_mean, p_std):
    params = {
        'weight_r_yom': weight_r_yom,
        'weight_r_ylm': weight_r_ylm,
        'weight_r_yfm': weight_r_yfm,
        'weight_r_yvm': weight_r_yvm,
        'bias_b0_yom': bias_b0_yom,
        'weight_b1_yom': weight_b1_yom,
        'bias_b0_ylm': bias_b0_ylm,
        'weight_b2_ylm': weight_b2_ylm,
        'bias_b0_yrm': bias_b0_yrm,
    }
    return _forward(x, y_obs, params, p_mean, p_std,
                    time_lag=128, spin_len=128, train_len=4096)


# speedup vs baseline: 1.0444x; 1.0444x over previous
import functools

import jax
import jax.numpy as jnp
from jax import lax
from jax.experimental import pallas as pl
from jax.experimental.pallas import tpu as pltpu

_ML = 2.9086
_SL = 1.898
_SCALE_MR = 500.0
_INV_SCALE_MR = 1.0 / _SCALE_MR
_LANES = 128

_COL_H = 0
_COL_C = 1
_COL_L = 2
_COL_LC = 3
_COL_BP = 4
_COL_IB = 5
_COL_OO = 6
_COL_OL = 7
_COL_OLC = 8
_COL_F = 9
_COL_STD = 10
_COL_OV = 11

(_P_HOO1, _P_G1, _P_KOOH, _P_AOOH, _P_SIG, _P_EXP, _P_THR, _P_STD) = range(8)
_N_PARAMS = 8

_GROUP = 128


def _round_up(x, m):
    return (x + m - 1) // m * m


def _extract_kernel(x_ref, p2_ref, u1_ref, u2_ref, ol_ref):
    rows = x_ref.shape[0]
    ol1 = jnp.full((1, _LANES), p2_ref[0], dtype=jnp.float32)
    k_ol = jnp.full((1, _LANES), p2_ref[1], dtype=jnp.float32)
    a_ol = jnp.full((1, _LANES), p2_ref[2], dtype=jnp.float32)
    for g in range(rows // _LANES):
        t = x_ref[pl.ds(g * _LANES, _LANES), 0:8].T
        u2r = t[1:2, :]
        u1_ref[pl.ds(g, 1), :] = t[0:1, :]
        u2_ref[pl.ds(g, 1), :] = u2r
        ol_ref[pl.ds(g, 1), :] = ol1 * jax.nn.sigmoid(k_ol + u2r * a_ol)


def _rnn_kernel(u1_ref, u2_ref, ol_ref, p_ref, out_ref, c_state, scr_ref, *,
                batch, seq_len, time_lag):

    shape = (1, _LANES)

    def splat(j):
        return jnp.full(shape, p_ref[j], dtype=jnp.float32)

    hoo1 = splat(_P_HOO1)
    g1 = splat(_P_G1)
    koo_h = splat(_P_KOOH)
    aoo_h = splat(_P_AOOH)
    sig = splat(_P_SIG)
    exp_yrm = splat(_P_EXP)
    thr = splat(_P_THR)
    obs_std = splat(_P_STD)
    lane = lax.broadcasted_iota(jnp.int32, shape, 1)
    _used = (_COL_H, _COL_C, _COL_L, _COL_LC, _COL_OO, _COL_OL,
             _COL_OLC, _COL_F, _COL_STD, _COL_OV)
    onehot = {j: (lane == j).astype(jnp.float32) for j in _used}

    def step(c0pair, u1, u2, ol):
        c0a, c0b = c0pair
        c0 = jnp.maximum(c0a, c0b)
        cpos = c0 > 0.0
        ta = jnp.tanh(koo_h + c0a * aoo_h)
        tb = jnp.tanh(koo_h + c0b * aoo_h)
        t = jnp.where(c0a >= c0b, ta, tb)
        r = pl.reciprocal(c0, approx=True)
        d = c0 - thr
        ad = jnp.abs(d)
        cm = c0 - ad
        e = u1 - sig * d
        q = ol * c0
        olc_c0 = jnp.where(cpos, jnp.minimum(q, u2), q)
        olc = jnp.where(cpos, jnp.minimum(ol, u2 * r), ol)
        hc = hoo1 * c0
        hcm = hoo1 * cm
        k1 = (g1 * c0 + e) - olc_c0
        k2 = g1 * cm + u1
        c1a = k1 - t * hc
        c1b = (k2 - t * hcm) - olc * cm
        return (c1a, c1b), t, olc, q, olc_c0

    out_ref[:, pl.ds(0, time_lag)] = jnp.zeros((_LANES, time_lag),
                                               out_ref.dtype)

    def row_body(g, k, cp0):
        row_off = (g * _GROUP + k) * seq_len
        cp = cp0
        for t in range(seq_len - 1):
            cp = step(cp, u1_ref[row_off + t], u2_ref[row_off + t],
                      ol_ref[row_off + t])[0]
        idx = row_off + seq_len - 1
        ol = ol_ref[idx]
        c = jnp.maximum(cp[0], cp[1])
        cp_new, t, olc, q, olc_c0 = step(cp, u1_ref[idx], u2_ref[idx], ol)
        a1 = hoo1 * t
        oo = hoo1 + a1
        f = (g1 - a1) - olc
        sgn = jnp.sign(c * _INV_SCALE_MR - exp_yrm)
        ov = jnp.minimum(sig * sgn, f)
        packed = ((oo * c) * onehot[_COL_H]
                  + c * onehot[_COL_C]
                  + q * onehot[_COL_L]
                  + olc_c0 * onehot[_COL_LC]
                  + oo * onehot[_COL_OO]
                  + ol * onehot[_COL_OL]
                  + olc * onehot[_COL_OLC]
                  + f * onehot[_COL_F]
                  + obs_std * onehot[_COL_STD]
                  + ov * onehot[_COL_OV])
        scr_ref[pl.ds(k, 1), :] = packed
        return cp_new

    def group_body(g, cp0):
        cp1 = lax.fori_loop(0, _GROUP,
                            lambda k, cp: row_body(g, k, cp), cp0, unroll=4)
        out_ref[:, pl.ds(g * _GROUP, _GROUP)] = scr_ref[...].T
        return cp1

    zero = jnp.zeros(shape, jnp.float32)
    cp_final = lax.fori_loop(time_lag // _GROUP, batch // _GROUP,
                             group_body, (zero, zero))
    c_state[...] = jnp.maximum(cp_final[0], cp_final[1])


def _forward(x, y_obs, params, p_mean, p_std, *, time_lag, spin_len,
             train_len):
    batch, seq, _ = x.shape
    x = x.astype(jnp.float32)

    f32 = lambda v: jnp.asarray(v, jnp.float32).reshape(())
    w_r_yom = f32(params['weight_r_yom'])
    w_r_ylm = f32(params['weight_r_ylm'])
    w_r_yfm = f32(params['weight_r_yfm'])
    w_r_yvm = f32(params['weight_r_yvm'])
    b0_yom = f32(params['bias_b0_yom'])
    w_b1_yom = f32(params['weight_b1_yom'])
    b0_ylm = f32(params['bias_b0_ylm'])
    w_b2_ylm = f32(params['weight_b2_ylm'])
    b0_yrm = f32(params['bias_b0_yrm'])
    mo = f32(p_mean)
    so = f32(p_std)

    e_o, e_l, e_f = jnp.exp(w_r_yom), jnp.exp(w_r_ylm), jnp.exp(w_r_yfm)
    denom = e_o + e_l + e_f
    oo1 = e_o / denom
    ol1 = e_l / denom
    sig_yvm = jax.nn.sigmoid(w_r_yvm)
    exp_yrm = jnp.exp(b0_yrm)
    thr = exp_yrm * jnp.float32(_SCALE_MR)
    a_oo = w_b1_yom / so
    k_oo = b0_yom - mo * a_oo
    a_ol = w_b2_ylm / jnp.float32(_SL)
    k_ol = b0_ylm - jnp.float32(_ML) * a_ol
    obs_std = jnp.std(y_obs[spin_len:train_len].astype(jnp.float32), ddof=1)

    n = batch * seq
    x2 = x.reshape(n, _LANES)
    p2_vec = jnp.stack([ol1, k_ol, a_ol]).astype(jnp.float32)
    npar = 2
    nblk = max(1, n // (npar * 2048))
    rows_blk = n // (npar * nblk)
    u1_2d, u2_2d, ol_2d = pl.pallas_call(
        _extract_kernel,
        out_shape=[jax.ShapeDtypeStruct((n // _LANES, _LANES), jnp.float32)
                   for _ in range(3)],
        grid_spec=pltpu.PrefetchScalarGridSpec(
            num_scalar_prefetch=0,
            grid=(npar, nblk),
            in_specs=[
                pl.BlockSpec((rows_blk, _LANES),
                             lambda i, j: (i * nblk + j, 0)),
                pl.BlockSpec(memory_space=pltpu.MemorySpace.SMEM),
            ],
            out_specs=[
                pl.BlockSpec((rows_blk // _LANES, _LANES),
                             lambda i, j: (i * nblk + j, 0))
                for _ in range(3)
            ],
        ),
        compiler_params=pltpu.CompilerParams(
            dimension_semantics=("parallel", "arbitrary")),
    )(x2, p2_vec)
    u1 = u1_2d.reshape(-1)
    u2 = u2_2d.reshape(-1)
    ol_all = ol_2d.reshape(-1)

    hoo1 = 0.5 * oo1
    p_vec = jnp.stack([hoo1, 1.0 - hoo1, 0.5 * k_oo, 0.5 * a_oo, sig_yvm,
                       exp_yrm, thr, obs_std]).astype(jnp.float32)

    _kernel_fn = functools.partial(_rnn_kernel, batch=batch, seq_len=seq,
                                   time_lag=time_lag)

    out = pl.pallas_call(
        _kernel_fn,
        out_shape=jax.ShapeDtypeStruct((_LANES, batch), jnp.float32),
        grid_spec=pltpu.PrefetchScalarGridSpec(
            num_scalar_prefetch=0,
            grid=(1,),
            in_specs=[
                pl.BlockSpec(memory_space=pltpu.MemorySpace.SMEM),
                pl.BlockSpec(memory_space=pltpu.MemorySpace.SMEM),
                pl.BlockSpec(memory_space=pltpu.MemorySpace.SMEM),
                pl.BlockSpec(memory_space=pltpu.MemorySpace.SMEM),
            ],
            out_specs=pl.BlockSpec((_LANES, batch), lambda i: (0, 0)),
            scratch_shapes=[pltpu.VMEM((1, _LANES), jnp.float32),
                            pltpu.VMEM((_GROUP, _LANES), jnp.float32)],
        ),
        compiler_params=pltpu.CompilerParams(
            dimension_semantics=("arbitrary",)),
    )(u1, u2, ol_all, p_vec)

    col = lambda j: out[j].reshape(batch, 1)
    h_n = col(_COL_H)
    obs_std_col = col(_COL_STD)
    h_nout = jnp.concatenate([h_n, obs_std_col], axis=1)
    return (h_n, col(_COL_C), col(_COL_L), col(_COL_LC), col(_COL_BP),
            col(_COL_IB), col(_COL_OO), col(_COL_OL), col(_COL_OLC),
            col(_COL_F), h_nout, obs_std_col, col(_COL_OV))


def kernel(x, y_obs, weight_r_yom, weight_r_ylm, weight_r_yfm, weight_r_yvm,
           bias_b0_yom, weight_b1_yom, bias_b0_ylm, weight_b2_ylm,
           bias_b0_yrm, p_mean, p_std):
    params = {
        'weight_r_yom': weight_r_yom,
        'weight_r_ylm': weight_r_ylm,
        'weight_r_yfm': weight_r_yfm,
        'weight_r_yvm': weight_r_yvm,
        'bias_b0_yom': bias_b0_yom,
        'weight_b1_yom': weight_b1_yom,
        'bias_b0_ylm': bias_b0_ylm,
        'weight_b2_ylm': weight_b2_ylm,
        'bias_b0_yrm': bias_b0_yrm,
    }
    return _forward(x, y_obs, params, p_mean, p_std,
                    time_lag=128, spin_len=128, train_len=4096)


# speedup vs baseline: 1.0583x; 1.0133x over previous
import functools

import jax
import jax.numpy as jnp
from jax import lax
from jax.experimental import pallas as pl
from jax.experimental.pallas import tpu as pltpu

_ML = 2.9086
_SL = 1.898
_SCALE_MR = 500.0
_INV_SCALE_MR = 1.0 / _SCALE_MR
_LANES = 128

_COL_H = 0
_COL_C = 1
_COL_L = 2
_COL_LC = 3
_COL_BP = 4
_COL_IB = 5
_COL_OO = 6
_COL_OL = 7
_COL_OLC = 8
_COL_F = 9
_COL_STD = 10
_COL_OV = 11

(_P_HOO1, _P_G1, _P_KOOH, _P_AOOH, _P_SIG, _P_EXP, _P_THR, _P_STD) = range(8)
_N_PARAMS = 8

_GROUP = 128


def _round_up(x, m):
    return (x + m - 1) // m * m


def _extract_kernel(x_ref, p2_ref, u1_ref, u2_ref, ol_ref):
    rows = x_ref.shape[0]
    ol1 = jnp.full((1, _LANES), p2_ref[0], dtype=jnp.float32)
    k_ol = jnp.full((1, _LANES), p2_ref[1], dtype=jnp.float32)
    a_ol = jnp.full((1, _LANES), p2_ref[2], dtype=jnp.float32)
    for g in range(rows // _LANES):
        t = x_ref[pl.ds(g * _LANES, _LANES), 0:8].T
        u2r = t[1:2, :]
        u1_ref[pl.ds(g, 1), :] = t[0:1, :]
        u2_ref[pl.ds(g, 1), :] = u2r
        ol_ref[pl.ds(g, 1), :] = ol1 * jax.nn.sigmoid(k_ol + u2r * a_ol)


def _rnn_kernel(u1_ref, u2_ref, ol_ref, p_ref, out_ref, c_state, scr_ref, *,
                batch, seq_len, time_lag):

    shape = (1, _LANES)

    def splat(j):
        return jnp.full(shape, p_ref[j], dtype=jnp.float32)

    hoo1 = splat(_P_HOO1)
    g1 = splat(_P_G1)
    koo_h = splat(_P_KOOH)
    aoo_h = splat(_P_AOOH)
    sig = splat(_P_SIG)
    exp_yrm = splat(_P_EXP)
    thr = splat(_P_THR)
    obs_std = splat(_P_STD)
    lane = lax.broadcasted_iota(jnp.int32, shape, 1)
    _used = (_COL_H, _COL_C, _COL_L, _COL_LC, _COL_OO, _COL_OL,
             _COL_OLC, _COL_F, _COL_STD, _COL_OV)
    onehot = {j: (lane == j).astype(jnp.float32) for j in _used}

    def step(c0pair, u1, u2, ol):
        c0a, c0b = c0pair
        c0 = jnp.maximum(c0a, c0b)
        cpos = c0 > 0.0
        ta = jnp.tanh(koo_h + c0a * aoo_h)
        tb = jnp.tanh(koo_h + c0b * aoo_h)
        t = jnp.where(c0a >= c0b, ta, tb)
        r = pl.reciprocal(c0, approx=True)
        d = c0 - thr
        ad = jnp.abs(d)
        cm = c0 - ad
        e = u1 - sig * d
        q = ol * c0
        olc_c0 = jnp.where(cpos, jnp.minimum(q, u2), q)
        olc = jnp.where(cpos, jnp.minimum(ol, u2 * r), ol)
        hc = hoo1 * c0
        hcm = hoo1 * cm
        k1 = (g1 * c0 + e) - olc_c0
        k2 = g1 * cm + u1
        c1a = k1 - t * hc
        c1b = (k2 - t * hcm) - olc * cm
        return (c1a, c1b), t, olc, q, olc_c0

    out_ref[:, pl.ds(0, time_lag)] = jnp.zeros((_LANES, time_lag),
                                               out_ref.dtype)

    def row_body(g, k, cp0):
        row_off = (g * _GROUP + k) * seq_len
        cp = cp0
        for t in range(seq_len - 1):
            cp = step(cp, u1_ref[row_off + t], u2_ref[row_off + t],
                      ol_ref[row_off + t])[0]
        idx = row_off + seq_len - 1
        ol = ol_ref[idx]
        c = jnp.maximum(cp[0], cp[1])
        cp_new, t, olc, q, olc_c0 = step(cp, u1_ref[idx], u2_ref[idx], ol)
        a1 = hoo1 * t
        oo = hoo1 + a1
        f = (g1 - a1) - olc
        sgn = jnp.sign(c * _INV_SCALE_MR - exp_yrm)
        ov = jnp.minimum(sig * sgn, f)
        packed = ((oo * c) * onehot[_COL_H]
                  + c * onehot[_COL_C]
                  + q * onehot[_COL_L]
                  + olc_c0 * onehot[_COL_LC]
                  + oo * onehot[_COL_OO]
                  + ol * onehot[_COL_OL]
                  + olc * onehot[_COL_OLC]
                  + f * onehot[_COL_F]
                  + obs_std * onehot[_COL_STD]
                  + ov * onehot[_COL_OV])
        scr_ref[pl.ds(k, 1), :] = packed
        return cp_new

    def group_body(g, cp0):
        cp1 = lax.fori_loop(0, _GROUP,
                            lambda k, cp: row_body(g, k, cp), cp0, unroll=8)
        out_ref[:, pl.ds(g * _GROUP, _GROUP)] = scr_ref[...].T
        return cp1

    zero = jnp.zeros(shape, jnp.float32)
    cp_final = lax.fori_loop(time_lag // _GROUP, batch // _GROUP,
                             group_body, (zero, zero))
    c_state[...] = jnp.maximum(cp_final[0], cp_final[1])


def _forward(x, y_obs, params, p_mean, p_std, *, time_lag, spin_len,
             train_len):
    batch, seq, _ = x.shape
    x = x.astype(jnp.float32)

    f32 = lambda v: jnp.asarray(v, jnp.float32).reshape(())
    w_r_yom = f32(params['weight_r_yom'])
    w_r_ylm = f32(params['weight_r_ylm'])
    w_r_yfm = f32(params['weight_r_yfm'])
    w_r_yvm = f32(params['weight_r_yvm'])
    b0_yom = f32(params['bias_b0_yom'])
    w_b1_yom = f32(params['weight_b1_yom'])
    b0_ylm = f32(params['bias_b0_ylm'])
    w_b2_ylm = f32(params['weight_b2_ylm'])
    b0_yrm = f32(params['bias_b0_yrm'])
    mo = f32(p_mean)
    so = f32(p_std)

    e_o, e_l, e_f = jnp.exp(w_r_yom), jnp.exp(w_r_ylm), jnp.exp(w_r_yfm)
    denom = e_o + e_l + e_f
    oo1 = e_o / denom
    ol1 = e_l / denom
    sig_yvm = jax.nn.sigmoid(w_r_yvm)
    exp_yrm = jnp.exp(b0_yrm)
    thr = exp_yrm * jnp.float32(_SCALE_MR)
    a_oo = w_b1_yom / so
    k_oo = b0_yom - mo * a_oo
    a_ol = w_b2_ylm / jnp.float32(_SL)
    k_ol = b0_ylm - jnp.float32(_ML) * a_ol
    obs_std = jnp.std(y_obs[spin_len:train_len].astype(jnp.float32), ddof=1)

    n = batch * seq
    x2 = x.reshape(n, _LANES)
    p2_vec = jnp.stack([ol1, k_ol, a_ol]).astype(jnp.float32)
    npar = 2
    nblk = max(1, n // (npar * 2048))
    rows_blk = n // (npar * nblk)
    u1_2d, u2_2d, ol_2d = pl.pallas_call(
        _extract_kernel,
        out_shape=[jax.ShapeDtypeStruct((n // _LANES, _LANES), jnp.float32)
                   for _ in range(3)],
        grid_spec=pltpu.PrefetchScalarGridSpec(
            num_scalar_prefetch=0,
            grid=(npar, nblk),
            in_specs=[
                pl.BlockSpec((rows_blk, _LANES),
                             lambda i, j: (i * nblk + j, 0)),
                pl.BlockSpec(memory_space=pltpu.MemorySpace.SMEM),
            ],
            out_specs=[
                pl.BlockSpec((rows_blk // _LANES, _LANES),
                             lambda i, j: (i * nblk + j, 0))
                for _ in range(3)
            ],
        ),
        compiler_params=pltpu.CompilerParams(
            dimension_semantics=("parallel", "arbitrary")),
    )(x2, p2_vec)
    u1 = u1_2d.reshape(-1)
    u2 = u2_2d.reshape(-1)
    ol_all = ol_2d.reshape(-1)

    hoo1 = 0.5 * oo1
    p_vec = jnp.stack([hoo1, 1.0 - hoo1, 0.5 * k_oo, 0.5 * a_oo, sig_yvm,
                       exp_yrm, thr, obs_std]).astype(jnp.float32)

    _kernel_fn = functools.partial(_rnn_kernel, batch=batch, seq_len=seq,
                                   time_lag=time_lag)

    out = pl.pallas_call(
        _kernel_fn,
        out_shape=jax.ShapeDtypeStruct((_LANES, batch), jnp.float32),
        grid_spec=pltpu.PrefetchScalarGridSpec(
            num_scalar_prefetch=0,
            grid=(1,),
            in_specs=[
                pl.BlockSpec(memory_space=pltpu.MemorySpace.SMEM),
                pl.BlockSpec(memory_space=pltpu.MemorySpace.SMEM),
                pl.BlockSpec(memory_space=pltpu.MemorySpace.SMEM),
                pl.BlockSpec(memory_space=pltpu.MemorySpace.SMEM),
            ],
            out_specs=pl.BlockSpec((_LANES, batch), lambda i: (0, 0)),
            scratch_shapes=[pltpu.VMEM((1, _LANES), jnp.float32),
                            pltpu.VMEM((_GROUP, _LANES), jnp.float32)],
        ),
        compiler_params=pltpu.CompilerParams(
            dimension_semantics=("arbitrary",)),
    )(u1, u2, ol_all, p_vec)

    col = lambda j: out[j].reshape(batch, 1)
    h_n = col(_COL_H)
    obs_std_col = col(_COL_STD)
    h_nout = jnp.concatenate([h_n, obs_std_col], axis=1)
    return (h_n, col(_COL_C), col(_COL_L), col(_COL_LC), col(_COL_BP),
            col(_COL_IB), col(_COL_OO), col(_COL_OL), col(_COL_OLC),
            col(_COL_F), h_nout, obs_std_col, col(_COL_OV))


def kernel(x, y_obs, weight_r_yom, weight_r_ylm, weight_r_yfm, weight_r_yvm,
           bias_b0_yom, weight_b1_yom, bias_b0_ylm, weight_b2_ylm,
           bias_b0_yrm, p_mean, p_std):
    params = {
        'weight_r_yom': weight_r_yom,
        'weight_r_ylm': weight_r_ylm,
        'weight_r_yfm': weight_r_yfm,
        'weight_r_yvm': weight_r_yvm,
        'bias_b0_yom': bias_b0_yom,
        'weight_b1_yom': weight_b1_yom,
        'bias_b0_ylm': bias_b0_ylm,
        'weight_b2_ylm': weight_b2_ylm,
        'bias_b0_yrm': bias_b0_yrm,
    }
    return _forward(x, y_obs, params, p_mean, p_std,
                    time_lag=128, spin_len=128, train_len=4096)


# speedup vs baseline: 1.0653x; 1.0067x over previous
import functools

import jax
import jax.numpy as jnp
from jax import lax
from jax.experimental import pallas as pl
from jax.experimental.pallas import tpu as pltpu

_ML = 2.9086
_SL = 1.898
_SCALE_MR = 500.0
_INV_SCALE_MR = 1.0 / _SCALE_MR
_LANES = 128

_COL_H = 0
_COL_C = 1
_COL_L = 2
_COL_LC = 3
_COL_BP = 4
_COL_IB = 5
_COL_OO = 6
_COL_OL = 7
_COL_OLC = 8
_COL_F = 9
_COL_STD = 10
_COL_OV = 11

(_P_HOO1, _P_G1, _P_KOOH, _P_AOOH, _P_SIG, _P_EXP, _P_THR, _P_STD) = range(8)
_N_PARAMS = 8

_GROUP = 128


def _round_up(x, m):
    return (x + m - 1) // m * m


def _extract_kernel(x_ref, p2_ref, u1_ref, u2_ref, ol_ref):
    rows = x_ref.shape[0]
    ol1 = jnp.full((1, _LANES), p2_ref[0], dtype=jnp.float32)
    k_ol = jnp.full((1, _LANES), p2_ref[1], dtype=jnp.float32)
    a_ol = jnp.full((1, _LANES), p2_ref[2], dtype=jnp.float32)
    for g in range(rows // _LANES):
        t = x_ref[pl.ds(g * _LANES, _LANES), 0:8].T
        u2r = t[1:2, :]
        u1_ref[pl.ds(g, 1), :] = t[0:1, :]
        u2_ref[pl.ds(g, 1), :] = u2r
        ol_ref[pl.ds(g, 1), :] = ol1 * jax.nn.sigmoid(k_ol + u2r * a_ol)


def _rnn_kernel(u1_ref, u2_ref, ol_ref, p_ref, out_ref, c_state, scr_ref, *,
                batch, seq_len, time_lag):

    shape = (1, _LANES)

    def splat(j):
        return jnp.full(shape, p_ref[j], dtype=jnp.float32)

    hoo1 = splat(_P_HOO1)
    g1 = splat(_P_G1)
    koo_h = splat(_P_KOOH)
    aoo_h = splat(_P_AOOH)
    sig = splat(_P_SIG)
    exp_yrm = splat(_P_EXP)
    thr = splat(_P_THR)
    obs_std = splat(_P_STD)
    lane = lax.broadcasted_iota(jnp.int32, shape, 1)
    _used = (_COL_H, _COL_C, _COL_L, _COL_LC, _COL_OO, _COL_OL,
             _COL_OLC, _COL_F, _COL_STD, _COL_OV)
    onehot = {j: (lane == j).astype(jnp.float32) for j in _used}

    def step(c0pair, u1, u2, ol):
        c0a, c0b = c0pair
        c0 = jnp.maximum(c0a, c0b)
        cpos = c0 > 0.0
        ta = jnp.tanh(koo_h + c0a * aoo_h)
        tb = jnp.tanh(koo_h + c0b * aoo_h)
        t = jnp.where(c0a >= c0b, ta, tb)
        r = pl.reciprocal(c0, approx=True)
        d = c0 - thr
        ad = jnp.abs(d)
        cm = c0 - ad
        e = u1 - sig * d
        q = ol * c0
        olc_c0 = jnp.where(cpos, jnp.minimum(q, u2), q)
        olc = jnp.where(cpos, jnp.minimum(ol, u2 * r), ol)
        hc = hoo1 * c0
        hcm = hoo1 * cm
        k1 = (g1 * c0 + e) - olc_c0
        k2 = g1 * cm + u1
        c1a = k1 - t * hc
        c1b = (k2 - t * hcm) - olc * cm
        return (c1a, c1b), t, olc, q, olc_c0

    out_ref[:, pl.ds(0, time_lag)] = jnp.zeros((_LANES, time_lag),
                                               out_ref.dtype)

    def row_body(g, k, cp0):
        row_off = (g * _GROUP + k) * seq_len
        cp = cp0
        for t in range(seq_len - 1):
            cp = step(cp, u1_ref[row_off + t], u2_ref[row_off + t],
                      ol_ref[row_off + t])[0]
        idx = row_off + seq_len - 1
        ol = ol_ref[idx]
        c = jnp.maximum(cp[0], cp[1])
        cp_new, t, olc, q, olc_c0 = step(cp, u1_ref[idx], u2_ref[idx], ol)
        a1 = hoo1 * t
        oo = hoo1 + a1
        f = (g1 - a1) - olc
        sgn = jnp.sign(c * _INV_SCALE_MR - exp_yrm)
        ov = jnp.minimum(sig * sgn, f)
        packed = ((oo * c) * onehot[_COL_H]
                  + c * onehot[_COL_C]
                  + q * onehot[_COL_L]
                  + olc_c0 * onehot[_COL_LC]
                  + oo * onehot[_COL_OO]
                  + ol * onehot[_COL_OL]
                  + olc * onehot[_COL_OLC]
                  + f * onehot[_COL_F]
                  + obs_std * onehot[_COL_STD]
                  + ov * onehot[_COL_OV])
        scr_ref[pl.ds(k, 1), :] = packed
        return cp_new

    def group_body(g, cp0):
        cp1 = lax.fori_loop(0, _GROUP,
                            lambda k, cp: row_body(g, k, cp), cp0, unroll=16)
        out_ref[:, pl.ds(g * _GROUP, _GROUP)] = scr_ref[...].T
        return cp1

    zero = jnp.zeros(shape, jnp.float32)
    cp_final = lax.fori_loop(time_lag // _GROUP, batch // _GROUP,
                             group_body, (zero, zero))
    c_state[...] = jnp.maximum(cp_final[0], cp_final[1])


def _forward(x, y_obs, params, p_mean, p_std, *, time_lag, spin_len,
             train_len):
    batch, seq, _ = x.shape
    x = x.astype(jnp.float32)

    f32 = lambda v: jnp.asarray(v, jnp.float32).reshape(())
    w_r_yom = f32(params['weight_r_yom'])
    w_r_ylm = f32(params['weight_r_ylm'])
    w_r_yfm = f32(params['weight_r_yfm'])
    w_r_yvm = f32(params['weight_r_yvm'])
    b0_yom = f32(params['bias_b0_yom'])
    w_b1_yom = f32(params['weight_b1_yom'])
    b0_ylm = f32(params['bias_b0_ylm'])
    w_b2_ylm = f32(params['weight_b2_ylm'])
    b0_yrm = f32(params['bias_b0_yrm'])
    mo = f32(p_mean)
    so = f32(p_std)

    e_o, e_l, e_f = jnp.exp(w_r_yom), jnp.exp(w_r_ylm), jnp.exp(w_r_yfm)
    denom = e_o + e_l + e_f
    oo1 = e_o / denom
    ol1 = e_l / denom
    sig_yvm = jax.nn.sigmoid(w_r_yvm)
    exp_yrm = jnp.exp(b0_yrm)
    thr = exp_yrm * jnp.float32(_SCALE_MR)
    a_oo = w_b1_yom / so
    k_oo = b0_yom - mo * a_oo
    a_ol = w_b2_ylm / jnp.float32(_SL)
    k_ol = b0_ylm - jnp.float32(_ML) * a_ol
    obs_std = jnp.std(y_obs[spin_len:train_len].astype(jnp.float32), ddof=1)

    n = batch * seq
    x2 = x.reshape(n, _LANES)
    p2_vec = jnp.stack([ol1, k_ol, a_ol]).astype(jnp.float32)
    npar = 2
    nblk = max(1, n // (npar * 2048))
    rows_blk = n // (npar * nblk)
    u1_2d, u2_2d, ol_2d = pl.pallas_call(
        _extract_kernel,
        out_shape=[jax.ShapeDtypeStruct((n // _LANES, _LANES), jnp.float32)
                   for _ in range(3)],
        grid_spec=pltpu.PrefetchScalarGridSpec(
            num_scalar_prefetch=0,
            grid=(npar, nblk),
            in_specs=[
                pl.BlockSpec((rows_blk, _LANES),
                             lambda i, j: (i * nblk + j, 0)),
                pl.BlockSpec(memory_space=pltpu.MemorySpace.SMEM),
            ],
            out_specs=[
                pl.BlockSpec((rows_blk // _LANES, _LANES),
                             lambda i, j: (i * nblk + j, 0))
                for _ in range(3)
            ],
        ),
        compiler_params=pltpu.CompilerParams(
            dimension_semantics=("parallel", "arbitrary")),
    )(x2, p2_vec)
    u1 = u1_2d.reshape(-1)
    u2 = u2_2d.reshape(-1)
    ol_all = ol_2d.reshape(-1)

    hoo1 = 0.5 * oo1
    p_vec = jnp.stack([hoo1, 1.0 - hoo1, 0.5 * k_oo, 0.5 * a_oo, sig_yvm,
                       exp_yrm, thr, obs_std]).astype(jnp.float32)

    _kernel_fn = functools.partial(_rnn_kernel, batch=batch, seq_len=seq,
                                   time_lag=time_lag)

    out = pl.pallas_call(
        _kernel_fn,
        out_shape=jax.ShapeDtypeStruct((_LANES, batch), jnp.float32),
        grid_spec=pltpu.PrefetchScalarGridSpec(
            num_scalar_prefetch=0,
            grid=(1,),
            in_specs=[
                pl.BlockSpec(memory_space=pltpu.MemorySpace.SMEM),
                pl.BlockSpec(memory_space=pltpu.MemorySpace.SMEM),
                pl.BlockSpec(memory_space=pltpu.MemorySpace.SMEM),
                pl.BlockSpec(memory_space=pltpu.MemorySpace.SMEM),
            ],
            out_specs=pl.BlockSpec((_LANES, batch), lambda i: (0, 0)),
            scratch_shapes=[pltpu.VMEM((1, _LANES), jnp.float32),
                            pltpu.VMEM((_GROUP, _LANES), jnp.float32)],
        ),
        compiler_params=pltpu.CompilerParams(
            dimension_semantics=("arbitrary",)),
    )(u1, u2, ol_all, p_vec)

    col = lambda j: out[j].reshape(batch, 1)
    h_n = col(_COL_H)
    obs_std_col = col(_COL_STD)
    h_nout = jnp.concatenate([h_n, obs_std_col], axis=1)
    return (h_n, col(_COL_C), col(_COL_L), col(_COL_LC), col(_COL_BP),
            col(_COL_IB), col(_COL_OO), col(_COL_OL), col(_COL_OLC),
            col(_COL_F), h_nout, obs_std_col, col(_COL_OV))


def kernel(x, y_obs, weight_r_yom, weight_r_ylm, weight_r_yfm, weight_r_yvm,
           bias_b0_yom, weight_b1_yom, bias_b0_ylm, weight_b2_ylm,
           bias_b0_yrm, p_mean, p_std):
    params = {
        'weight_r_yom': weight_r_yom,
        'weight_r_ylm': weight_r_ylm,
        'weight_r_yfm': weight_r_yfm,
        'weight_r_yvm': weight_r_yvm,
        'bias_b0_yom': bias_b0_yom,
        'weight_b1_yom': weight_b1_yom,
        'bias_b0_ylm': bias_b0_ylm,
        'weight_b2_ylm': weight_b2_ylm,
        'bias_b0_yrm': bias_b0_yrm,
    }
    return _forward(x, y_obs, params, p_mean, p_std,
                    time_lag=128, spin_len=128, train_len=4096)


# speedup vs baseline: 1.0687x; 1.0031x over previous
import functools

import jax
import jax.numpy as jnp
from jax import lax
from jax.experimental import pallas as pl
from jax.experimental.pallas import tpu as pltpu

_ML = 2.9086
_SL = 1.898
_SCALE_MR = 500.0
_INV_SCALE_MR = 1.0 / _SCALE_MR
_LANES = 128

_COL_H = 0
_COL_C = 1
_COL_L = 2
_COL_LC = 3
_COL_BP = 4
_COL_IB = 5
_COL_OO = 6
_COL_OL = 7
_COL_OLC = 8
_COL_F = 9
_COL_STD = 10
_COL_OV = 11

(_P_HOO1, _P_G1, _P_KOOH, _P_AOOH, _P_SIG, _P_EXP, _P_THR, _P_STD) = range(8)
_N_PARAMS = 8

_GROUP = 128


def _round_up(x, m):
    return (x + m - 1) // m * m


def _extract_kernel(x_ref, p2_ref, u1_ref, u2_ref, ol_ref):
    rows = x_ref.shape[0]
    ol1 = jnp.full((1, _LANES), p2_ref[0], dtype=jnp.float32)
    k_ol = jnp.full((1, _LANES), p2_ref[1], dtype=jnp.float32)
    a_ol = jnp.full((1, _LANES), p2_ref[2], dtype=jnp.float32)
    for g in range(rows // _LANES):
        t = x_ref[pl.ds(g * _LANES, _LANES), 0:8].T
        u2r = t[1:2, :]
        u1_ref[pl.ds(g, 1), :] = t[0:1, :]
        u2_ref[pl.ds(g, 1), :] = u2r
        ol_ref[pl.ds(g, 1), :] = ol1 * jax.nn.sigmoid(k_ol + u2r * a_ol)


def _rnn_kernel(u1_ref, u2_ref, ol_ref, p_ref, out_ref, c_state, scr_ref, *,
                batch, seq_len, time_lag):

    shape = (1, _LANES)

    def splat(j):
        return jnp.full(shape, p_ref[j], dtype=jnp.float32)

    hoo1 = splat(_P_HOO1)
    g1 = splat(_P_G1)
    koo_h = splat(_P_KOOH)
    aoo_h = splat(_P_AOOH)
    sig = splat(_P_SIG)
    exp_yrm = splat(_P_EXP)
    thr = splat(_P_THR)
    obs_std = splat(_P_STD)
    lane = lax.broadcasted_iota(jnp.int32, shape, 1)
    _used = (_COL_H, _COL_C, _COL_L, _COL_LC, _COL_OO, _COL_OL,
             _COL_OLC, _COL_F, _COL_STD, _COL_OV)
    onehot = {j: (lane == j).astype(jnp.float32) for j in _used}

    def step(c0pair, u1, u2, ol):
        c0a, c0b = c0pair
        c0 = jnp.maximum(c0a, c0b)
        cpos = c0 > 0.0
        ta = jnp.tanh(koo_h + c0a * aoo_h)
        tb = jnp.tanh(koo_h + c0b * aoo_h)
        t = jnp.where(c0a >= c0b, ta, tb)
        r = pl.reciprocal(c0, approx=True)
        d = c0 - thr
        ad = jnp.abs(d)
        cm = c0 - ad
        e = u1 - sig * d
        q = ol * c0
        olc_c0 = jnp.where(cpos, jnp.minimum(q, u2), q)
        olc = jnp.where(cpos, jnp.minimum(ol, u2 * r), ol)
        hc = hoo1 * c0
        hcm = hoo1 * cm
        k1 = (g1 * c0 + e) - olc_c0
        k2 = g1 * cm + u1
        c1a = k1 - t * hc
        c1b = (k2 - t * hcm) - olc * cm
        return (c1a, c1b), t, olc, q, olc_c0

    out_ref[:, pl.ds(0, time_lag)] = jnp.zeros((_LANES, time_lag),
                                               out_ref.dtype)

    def row_body(g, k, cp0):
        row_off = (g * _GROUP + k) * seq_len
        cp = cp0
        for t in range(seq_len - 1):
            cp = step(cp, u1_ref[row_off + t], u2_ref[row_off + t],
                      ol_ref[row_off + t])[0]
        idx = row_off + seq_len - 1
        ol = ol_ref[idx]
        c = jnp.maximum(cp[0], cp[1])
        cp_new, t, olc, q, olc_c0 = step(cp, u1_ref[idx], u2_ref[idx], ol)
        a1 = hoo1 * t
        oo = hoo1 + a1
        f = (g1 - a1) - olc
        sgn = jnp.sign(c * _INV_SCALE_MR - exp_yrm)
        ov = jnp.minimum(sig * sgn, f)
        packed = ((oo * c) * onehot[_COL_H]
                  + c * onehot[_COL_C]
                  + q * onehot[_COL_L]
                  + olc_c0 * onehot[_COL_LC]
                  + oo * onehot[_COL_OO]
                  + ol * onehot[_COL_OL]
                  + olc * onehot[_COL_OLC]
                  + f * onehot[_COL_F]
                  + obs_std * onehot[_COL_STD]
                  + ov * onehot[_COL_OV])
        scr_ref[pl.ds(k, 1), :] = packed
        return cp_new

    def group_body(g, cp0):
        cp1 = lax.fori_loop(0, _GROUP,
                            lambda k, cp: row_body(g, k, cp), cp0, unroll=32)
        out_ref[:, pl.ds(g * _GROUP, _GROUP)] = scr_ref[...].T
        return cp1

    zero = jnp.zeros(shape, jnp.float32)
    cp_final = lax.fori_loop(time_lag // _GROUP, batch // _GROUP,
                             group_body, (zero, zero))
    c_state[...] = jnp.maximum(cp_final[0], cp_final[1])


def _forward(x, y_obs, params, p_mean, p_std, *, time_lag, spin_len,
             train_len):
    batch, seq, _ = x.shape
    x = x.astype(jnp.float32)

    f32 = lambda v: jnp.asarray(v, jnp.float32).reshape(())
    w_r_yom = f32(params['weight_r_yom'])
    w_r_ylm = f32(params['weight_r_ylm'])
    w_r_yfm = f32(params['weight_r_yfm'])
    w_r_yvm = f32(params['weight_r_yvm'])
    b0_yom = f32(params['bias_b0_yom'])
    w_b1_yom = f32(params['weight_b1_yom'])
    b0_ylm = f32(params['bias_b0_ylm'])
    w_b2_ylm = f32(params['weight_b2_ylm'])
    b0_yrm = f32(params['bias_b0_yrm'])
    mo = f32(p_mean)
    so = f32(p_std)

    e_o, e_l, e_f = jnp.exp(w_r_yom), jnp.exp(w_r_ylm), jnp.exp(w_r_yfm)
    denom = e_o + e_l + e_f
    oo1 = e_o / denom
    ol1 = e_l / denom
    sig_yvm = jax.nn.sigmoid(w_r_yvm)
    exp_yrm = jnp.exp(b0_yrm)
    thr = exp_yrm * jnp.float32(_SCALE_MR)
    a_oo = w_b1_yom / so
    k_oo = b0_yom - mo * a_oo
    a_ol = w_b2_ylm / jnp.float32(_SL)
    k_ol = b0_ylm - jnp.float32(_ML) * a_ol
    obs_std = jnp.std(y_obs[spin_len:train_len].astype(jnp.float32), ddof=1)

    n = batch * seq
    x2 = x.reshape(n, _LANES)
    p2_vec = jnp.stack([ol1, k_ol, a_ol]).astype(jnp.float32)
    npar = 2
    nblk = max(1, n // (npar * 2048))
    rows_blk = n // (npar * nblk)
    u1_2d, u2_2d, ol_2d = pl.pallas_call(
        _extract_kernel,
        out_shape=[jax.ShapeDtypeStruct((n // _LANES, _LANES), jnp.float32)
                   for _ in range(3)],
        grid_spec=pltpu.PrefetchScalarGridSpec(
            num_scalar_prefetch=0,
            grid=(npar, nblk),
            in_specs=[
                pl.BlockSpec((rows_blk, _LANES),
                             lambda i, j: (i * nblk + j, 0)),
                pl.BlockSpec(memory_space=pltpu.MemorySpace.SMEM),
            ],
            out_specs=[
                pl.BlockSpec((rows_blk // _LANES, _LANES),
                             lambda i, j: (i * nblk + j, 0))
                for _ in range(3)
            ],
        ),
        compiler_params=pltpu.CompilerParams(
            dimension_semantics=("parallel", "arbitrary")),
    )(x2, p2_vec)
    u1 = u1_2d.reshape(-1)
    u2 = u2_2d.reshape(-1)
    ol_all = ol_2d.reshape(-1)

    hoo1 = 0.5 * oo1
    p_vec = jnp.stack([hoo1, 1.0 - hoo1, 0.5 * k_oo, 0.5 * a_oo, sig_yvm,
                       exp_yrm, thr, obs_std]).astype(jnp.float32)

    _kernel_fn = functools.partial(_rnn_kernel, batch=batch, seq_len=seq,
                                   time_lag=time_lag)

    out = pl.pallas_call(
        _kernel_fn,
        out_shape=jax.ShapeDtypeStruct((_LANES, batch), jnp.float32),
        grid_spec=pltpu.PrefetchScalarGridSpec(
            num_scalar_prefetch=0,
            grid=(1,),
            in_specs=[
                pl.BlockSpec(memory_space=pltpu.MemorySpace.SMEM),
                pl.BlockSpec(memory_space=pltpu.MemorySpace.SMEM),
                pl.BlockSpec(memory_space=pltpu.MemorySpace.SMEM),
                pl.BlockSpec(memory_space=pltpu.MemorySpace.SMEM),
            ],
            out_specs=pl.BlockSpec((_LANES, batch), lambda i: (0, 0)),
            scratch_shapes=[pltpu.VMEM((1, _LANES), jnp.float32),
                            pltpu.VMEM((_GROUP, _LANES), jnp.float32)],
        ),
        compiler_params=pltpu.CompilerParams(
            dimension_semantics=("arbitrary",)),
    )(u1, u2, ol_all, p_vec)

    col = lambda j: out[j].reshape(batch, 1)
    h_n = col(_COL_H)
    obs_std_col = col(_COL_STD)
    h_nout = jnp.concatenate([h_n, obs_std_col], axis=1)
    return (h_n, col(_COL_C), col(_COL_L), col(_COL_LC), col(_COL_BP),
            col(_COL_IB), col(_COL_OO), col(_COL_OL), col(_COL_OLC),
            col(_COL_F), h_nout, obs_std_col, col(_COL_OV))


def kernel(x, y_obs, weight_r_yom, weight_r_ylm, weight_r_yfm, weight_r_yvm,
           bias_b0_yom, weight_b1_yom, bias_b0_ylm, weight_b2_ylm,
           bias_b0_yrm, p_mean, p_std):
    params = {
        'weight_r_yom': weight_r_yom,
        'weight_r_ylm': weight_r_ylm,
        'weight_r_yfm': weight_r_yfm,
        'weight_r_yvm': weight_r_yvm,
        'bias_b0_yom': bias_b0_yom,
        'weight_b1_yom': weight_b1_yom,
        'bias_b0_ylm': bias_b0_ylm,
        'weight_b2_ylm': weight_b2_ylm,
        'bias_b0_yrm': bias_b0_yrm,
    }
    return _forward(x, y_obs, params, p_mean, p_std,
                    time_lag=128, spin_len=128, train_len=4096)


# speedup vs baseline: 1.0855x; 1.0157x over previous
import functools

import jax
import jax.numpy as jnp
from jax import lax
from jax.experimental import pallas as pl
from jax.experimental.pallas import tpu as pltpu

_ML = 2.9086
_SL = 1.898
_SCALE_MR = 500.0
_INV_SCALE_MR = 1.0 / _SCALE_MR
_LANES = 128

_COL_H = 0
_COL_C = 1
_COL_L = 2
_COL_LC = 3
_COL_BP = 4
_COL_IB = 5
_COL_OO = 6
_COL_OL = 7
_COL_OLC = 8
_COL_F = 9
_COL_STD = 10
_COL_OV = 11

(_P_HOO1, _P_G1, _P_KOOH, _P_AOOH, _P_SIG, _P_EXP, _P_THR, _P_STD) = range(8)
_N_PARAMS = 8

_GROUP = 128


def _round_up(x, m):
    return (x + m - 1) // m * m


def _extract_kernel(x_ref, p2_ref, u1_ref, u2_ref, ol_ref):
    rows = x_ref.shape[0]
    ol1 = jnp.full((1, _LANES), p2_ref[0], dtype=jnp.float32)
    k_ol = jnp.full((1, _LANES), p2_ref[1], dtype=jnp.float32)
    a_ol = jnp.full((1, _LANES), p2_ref[2], dtype=jnp.float32)
    for g in range(rows // _LANES):
        t = x_ref[pl.ds(g * _LANES, _LANES), 0:8].T
        u2r = t[1:2, :]
        u1_ref[pl.ds(g, 1), :] = t[0:1, :]
        u2_ref[pl.ds(g, 1), :] = u2r
        ol_ref[pl.ds(g, 1), :] = ol1 * jax.nn.sigmoid(k_ol + u2r * a_ol)


def _rnn_kernel(u1_ref, u2_ref, ol_ref, p_ref, out_ref, c_state, scr_ref, *,
                batch, seq_len, time_lag):

    shape = (1, _LANES)

    def splat(j):
        return jnp.full(shape, p_ref[j], dtype=jnp.float32)

    hoo1 = splat(_P_HOO1)
    g1 = splat(_P_G1)
    koo_h = splat(_P_KOOH)
    aoo_h = splat(_P_AOOH)
    sig = splat(_P_SIG)
    exp_yrm = splat(_P_EXP)
    thr = splat(_P_THR)
    obs_std = splat(_P_STD)
    lane = lax.broadcasted_iota(jnp.int32, shape, 1)
    _used = (_COL_H, _COL_C, _COL_L, _COL_LC, _COL_OO, _COL_OL,
             _COL_OLC, _COL_F, _COL_STD, _COL_OV)
    onehot = {j: (lane == j).astype(jnp.float32) for j in _used}

    def step(c0pair, u1, u2, ol):
        c0a, c0b = c0pair
        c0 = jnp.maximum(c0a, c0b)
        cpos = c0 > 0.0
        ta = jnp.tanh(koo_h + c0a * aoo_h)
        tb = jnp.tanh(koo_h + c0b * aoo_h)
        t = jnp.where(c0a >= c0b, ta, tb)
        r = pl.reciprocal(c0, approx=True)
        d = c0 - thr
        ad = jnp.abs(d)
        cm = c0 - ad
        e = u1 - sig * d
        q = ol * c0
        olc_c0 = jnp.where(cpos, jnp.minimum(q, u2), q)
        olc = jnp.where(cpos, jnp.minimum(ol, u2 * r), ol)
        hc = hoo1 * c0
        hcm = hoo1 * cm
        k1 = (g1 * c0 + e) - olc_c0
        k2 = g1 * cm + u1
        c1a = k1 - t * hc
        c1b = (k2 - t * hcm) - olc * cm
        return (c1a, c1b), t, olc, q, olc_c0

    out_ref[:, pl.ds(0, time_lag)] = jnp.zeros((_LANES, time_lag),
                                               out_ref.dtype)

    def row_body(g, k, cp0):
        row_off = (g * _GROUP + k) * seq_len
        cp = cp0
        for t in range(seq_len - 1):
            cp = step(cp, u1_ref[row_off + t], u2_ref[row_off + t],
                      ol_ref[row_off + t])[0]
        idx = row_off + seq_len - 1
        ol = ol_ref[idx]
        c = jnp.maximum(cp[0], cp[1])
        cp_new, t, olc, q, olc_c0 = step(cp, u1_ref[idx], u2_ref[idx], ol)
        a1 = hoo1 * t
        oo = hoo1 + a1
        f = (g1 - a1) - olc
        sgn = jnp.sign(c * _INV_SCALE_MR - exp_yrm)
        ov = jnp.minimum(sig * sgn, f)
        packed = ((oo * c) * onehot[_COL_H]
                  + c * onehot[_COL_C]
                  + q * onehot[_COL_L]
                  + olc_c0 * onehot[_COL_LC]
                  + oo * onehot[_COL_OO]
                  + ol * onehot[_COL_OL]
                  + olc * onehot[_COL_OLC]
                  + f * onehot[_COL_F]
                  + obs_std * onehot[_COL_STD]
                  + ov * onehot[_COL_OV])
        scr_ref[pl.ds(k, 1), :] = packed
        return cp_new

    def group_body(g, cp0):
        cp1 = lax.fori_loop(0, _GROUP,
                            lambda k, cp: row_body(g, k, cp), cp0, unroll=64)
        out_ref[:, pl.ds(g * _GROUP, _GROUP)] = scr_ref[...].T
        return cp1

    zero = jnp.zeros(shape, jnp.float32)
    cp_final = lax.fori_loop(time_lag // _GROUP, batch // _GROUP,
                             group_body, (zero, zero))
    c_state[...] = jnp.maximum(cp_final[0], cp_final[1])


def _forward(x, y_obs, params, p_mean, p_std, *, time_lag, spin_len,
             train_len):
    batch, seq, _ = x.shape
    x = x.astype(jnp.float32)

    f32 = lambda v: jnp.asarray(v, jnp.float32).reshape(())
    w_r_yom = f32(params['weight_r_yom'])
    w_r_ylm = f32(params['weight_r_ylm'])
    w_r_yfm = f32(params['weight_r_yfm'])
    w_r_yvm = f32(params['weight_r_yvm'])
    b0_yom = f32(params['bias_b0_yom'])
    w_b1_yom = f32(params['weight_b1_yom'])
    b0_ylm = f32(params['bias_b0_ylm'])
    w_b2_ylm = f32(params['weight_b2_ylm'])
    b0_yrm = f32(params['bias_b0_yrm'])
    mo = f32(p_mean)
    so = f32(p_std)

    e_o, e_l, e_f = jnp.exp(w_r_yom), jnp.exp(w_r_ylm), jnp.exp(w_r_yfm)
    denom = e_o + e_l + e_f
    oo1 = e_o / denom
    ol1 = e_l / denom
    sig_yvm = jax.nn.sigmoid(w_r_yvm)
    exp_yrm = jnp.exp(b0_yrm)
    thr = exp_yrm * jnp.float32(_SCALE_MR)
    a_oo = w_b1_yom / so
    k_oo = b0_yom - mo * a_oo
    a_ol = w_b2_ylm / jnp.float32(_SL)
    k_ol = b0_ylm - jnp.float32(_ML) * a_ol
    obs_std = jnp.std(y_obs[spin_len:train_len].astype(jnp.float32), ddof=1)

    n = batch * seq
    x2 = x.reshape(n, _LANES)
    p2_vec = jnp.stack([ol1, k_ol, a_ol]).astype(jnp.float32)
    npar = 2
    nblk = max(1, n // (npar * 8192))
    rows_blk = n // (npar * nblk)
    u1_2d, u2_2d, ol_2d = pl.pallas_call(
        _extract_kernel,
        out_shape=[jax.ShapeDtypeStruct((n // _LANES, _LANES), jnp.float32)
                   for _ in range(3)],
        grid_spec=pltpu.PrefetchScalarGridSpec(
            num_scalar_prefetch=0,
            grid=(npar, nblk),
            in_specs=[
                pl.BlockSpec((rows_blk, _LANES),
                             lambda i, j: (i * nblk + j, 0)),
                pl.BlockSpec(memory_space=pltpu.MemorySpace.SMEM),
            ],
            out_specs=[
                pl.BlockSpec((rows_blk // _LANES, _LANES),
                             lambda i, j: (i * nblk + j, 0))
                for _ in range(3)
            ],
        ),
        compiler_params=pltpu.CompilerParams(
            dimension_semantics=("parallel", "arbitrary")),
    )(x2, p2_vec)
    u1 = u1_2d.reshape(-1)
    u2 = u2_2d.reshape(-1)
    ol_all = ol_2d.reshape(-1)

    hoo1 = 0.5 * oo1
    p_vec = jnp.stack([hoo1, 1.0 - hoo1, 0.5 * k_oo, 0.5 * a_oo, sig_yvm,
                       exp_yrm, thr, obs_std]).astype(jnp.float32)

    _kernel_fn = functools.partial(_rnn_kernel, batch=batch, seq_len=seq,
                                   time_lag=time_lag)

    out = pl.pallas_call(
        _kernel_fn,
        out_shape=jax.ShapeDtypeStruct((_LANES, batch), jnp.float32),
        grid_spec=pltpu.PrefetchScalarGridSpec(
            num_scalar_prefetch=0,
            grid=(1,),
            in_specs=[
                pl.BlockSpec(memory_space=pltpu.MemorySpace.SMEM),
                pl.BlockSpec(memory_space=pltpu.MemorySpace.SMEM),
                pl.BlockSpec(memory_space=pltpu.MemorySpace.SMEM),
                pl.BlockSpec(memory_space=pltpu.MemorySpace.SMEM),
            ],
            out_specs=pl.BlockSpec((_LANES, batch), lambda i: (0, 0)),
            scratch_shapes=[pltpu.VMEM((1, _LANES), jnp.float32),
                            pltpu.VMEM((_GROUP, _LANES), jnp.float32)],
        ),
        compiler_params=pltpu.CompilerParams(
            dimension_semantics=("arbitrary",)),
    )(u1, u2, ol_all, p_vec)

    col = lambda j: out[j].reshape(batch, 1)
    h_n = col(_COL_H)
    obs_std_col = col(_COL_STD)
    h_nout = jnp.concatenate([h_n, obs_std_col], axis=1)
    return (h_n, col(_COL_C), col(_COL_L), col(_COL_LC), col(_COL_BP),
            col(_COL_IB), col(_COL_OO), col(_COL_OL), col(_COL_OLC),
            col(_COL_F), h_nout, obs_std_col, col(_COL_OV))


def kernel(x, y_obs, weight_r_yom, weight_r_ylm, weight_r_yfm, weight_r_yvm,
           bias_b0_yom, weight_b1_yom, bias_b0_ylm, weight_b2_ylm,
           bias_b0_yrm, p_mean, p_std):
    params = {
        'weight_r_yom': weight_r_yom,
        'weight_r_ylm': weight_r_ylm,
        'weight_r_yfm': weight_r_yfm,
        'weight_r_yvm': weight_r_yvm,
        'bias_b0_yom': bias_b0_yom,
        'weight_b1_yom': weight_b1_yom,
        'bias_b0_ylm': bias_b0_ylm,
        'weight_b2_ylm': weight_b2_ylm,
        'bias_b0_yrm': bias_b0_yrm,
    }
    return _forward(x, y_obs, params, p_mean, p_std,
                    time_lag=128, spin_len=128, train_len=4096)


# speedup vs baseline: 1.0910x; 1.0051x over previous
import functools

import jax
import jax.numpy as jnp
from jax import lax
from jax.experimental import pallas as pl
from jax.experimental.pallas import tpu as pltpu

_ML = 2.9086
_SL = 1.898
_SCALE_MR = 500.0
_INV_SCALE_MR = 1.0 / _SCALE_MR
_LANES = 128

_COL_H = 0
_COL_C = 1
_COL_L = 2
_COL_LC = 3
_COL_BP = 4
_COL_IB = 5
_COL_OO = 6
_COL_OL = 7
_COL_OLC = 8
_COL_F = 9
_COL_STD = 10
_COL_OV = 11

(_P_HOO1, _P_G1, _P_KOOH, _P_AOOH, _P_SIG, _P_EXP, _P_THR, _P_STD) = range(8)
_N_PARAMS = 8

_GROUP = 128


def _round_up(x, m):
    return (x + m - 1) // m * m


def _extract_kernel(x_ref, p2_ref, u1_ref, u2_ref, ol_ref):
    rows = x_ref.shape[0]
    ol1 = jnp.full((1, _LANES), p2_ref[0], dtype=jnp.float32)
    k_ol = jnp.full((1, _LANES), p2_ref[1], dtype=jnp.float32)
    a_ol = jnp.full((1, _LANES), p2_ref[2], dtype=jnp.float32)
    for g in range(rows // _LANES):
        t = x_ref[pl.ds(g * _LANES, _LANES), 0:8].T
        u2r = t[1:2, :]
        u1_ref[pl.ds(g, 1), :] = t[0:1, :]
        u2_ref[pl.ds(g, 1), :] = u2r
        ol_ref[pl.ds(g, 1), :] = ol1 * jax.nn.sigmoid(k_ol + u2r * a_ol)


def _rnn_kernel(u1_ref, u2_ref, ol_ref, p_ref, out_ref, c_state, scr_ref, *,
                batch, seq_len, time_lag):

    shape = (1, _LANES)

    def splat(j):
        return jnp.full(shape, p_ref[j], dtype=jnp.float32)

    hoo1 = splat(_P_HOO1)
    g1 = splat(_P_G1)
    koo_h = splat(_P_KOOH)
    aoo_h = splat(_P_AOOH)
    sig = splat(_P_SIG)
    exp_yrm = splat(_P_EXP)
    thr = splat(_P_THR)
    obs_std = splat(_P_STD)
    lane = lax.broadcasted_iota(jnp.int32, shape, 1)
    _used = (_COL_H, _COL_C, _COL_L, _COL_LC, _COL_OO, _COL_OL,
             _COL_OLC, _COL_F, _COL_STD, _COL_OV)
    onehot = {j: (lane == j).astype(jnp.float32) for j in _used}

    def step(c0pair, u1, u2, ol):
        c0a, c0b = c0pair
        c0 = jnp.maximum(c0a, c0b)
        cpos = c0 > 0.0
        ta = jnp.tanh(koo_h + c0a * aoo_h)
        tb = jnp.tanh(koo_h + c0b * aoo_h)
        t = jnp.where(c0a >= c0b, ta, tb)
        r = pl.reciprocal(c0, approx=True)
        d = c0 - thr
        ad = jnp.abs(d)
        cm = c0 - ad
        e = u1 - sig * d
        q = ol * c0
        olc_c0 = jnp.where(cpos, jnp.minimum(q, u2), q)
        olc = jnp.where(cpos, jnp.minimum(ol, u2 * r), ol)
        hc = hoo1 * c0
        hcm = hoo1 * cm
        k1 = (g1 * c0 + e) - olc_c0
        k2 = g1 * cm + u1
        c1a = k1 - t * hc
        c1b = (k2 - t * hcm) - olc * cm
        return (c1a, c1b), t, olc, q, olc_c0

    out_ref[:, pl.ds(0, time_lag)] = jnp.zeros((_LANES, time_lag),
                                               out_ref.dtype)

    def row_body(g, k, cp0):
        row_off = (g * _GROUP + k) * seq_len
        cp = cp0
        for t in range(seq_len - 1):
            cp = step(cp, u1_ref[row_off + t], u2_ref[row_off + t],
                      ol_ref[row_off + t])[0]
        idx = row_off + seq_len - 1
        ol = ol_ref[idx]
        c = jnp.maximum(cp[0], cp[1])
        cp_new, t, olc, q, olc_c0 = step(cp, u1_ref[idx], u2_ref[idx], ol)
        a1 = hoo1 * t
        oo = hoo1 + a1
        f = (g1 - a1) - olc
        sgn = jnp.sign(c * _INV_SCALE_MR - exp_yrm)
        ov = jnp.minimum(sig * sgn, f)
        packed = ((oo * c) * onehot[_COL_H]
                  + c * onehot[_COL_C]
                  + q * onehot[_COL_L]
                  + olc_c0 * onehot[_COL_LC]
                  + oo * onehot[_COL_OO]
                  + ol * onehot[_COL_OL]
                  + olc * onehot[_COL_OLC]
                  + f * onehot[_COL_F]
                  + obs_std * onehot[_COL_STD]
                  + ov * onehot[_COL_OV])
        scr_ref[pl.ds(k, 1), :] = packed
        return cp_new

    def group_body(g, cp0):
        cp1 = lax.fori_loop(0, _GROUP,
                            lambda k, cp: row_body(g, k, cp), cp0, unroll=128)
        out_ref[:, pl.ds(g * _GROUP, _GROUP)] = scr_ref[...].T
        return cp1

    zero = jnp.zeros(shape, jnp.float32)
    cp_final = lax.fori_loop(time_lag // _GROUP, batch // _GROUP,
                             group_body, (zero, zero))
    c_state[...] = jnp.maximum(cp_final[0], cp_final[1])


def _forward(x, y_obs, params, p_mean, p_std, *, time_lag, spin_len,
             train_len):
    batch, seq, _ = x.shape
    x = x.astype(jnp.float32)

    f32 = lambda v: jnp.asarray(v, jnp.float32).reshape(())
    w_r_yom = f32(params['weight_r_yom'])
    w_r_ylm = f32(params['weight_r_ylm'])
    w_r_yfm = f32(params['weight_r_yfm'])
    w_r_yvm = f32(params['weight_r_yvm'])
    b0_yom = f32(params['bias_b0_yom'])
    w_b1_yom = f32(params['weight_b1_yom'])
    b0_ylm = f32(params['bias_b0_ylm'])
    w_b2_ylm = f32(params['weight_b2_ylm'])
    b0_yrm = f32(params['bias_b0_yrm'])
    mo = f32(p_mean)
    so = f32(p_std)

    e_o, e_l, e_f = jnp.exp(w_r_yom), jnp.exp(w_r_ylm), jnp.exp(w_r_yfm)
    denom = e_o + e_l + e_f
    oo1 = e_o / denom
    ol1 = e_l / denom
    sig_yvm = jax.nn.sigmoid(w_r_yvm)
    exp_yrm = jnp.exp(b0_yrm)
    thr = exp_yrm * jnp.float32(_SCALE_MR)
    a_oo = w_b1_yom / so
    k_oo = b0_yom - mo * a_oo
    a_ol = w_b2_ylm / jnp.float32(_SL)
    k_ol = b0_ylm - jnp.float32(_ML) * a_ol
    obs_std = jnp.std(y_obs[spin_len:train_len].astype(jnp.float32), ddof=1)

    n = batch * seq
    x2 = x.reshape(n, _LANES)
    p2_vec = jnp.stack([ol1, k_ol, a_ol]).astype(jnp.float32)
    npar = 2
    nblk = max(1, n // (npar * 8192))
    rows_blk = n // (npar * nblk)
    u1_2d, u2_2d, ol_2d = pl.pallas_call(
        _extract_kernel,
        out_shape=[jax.ShapeDtypeStruct((n // _LANES, _LANES), jnp.float32)
                   for _ in range(3)],
        grid_spec=pltpu.PrefetchScalarGridSpec(
            num_scalar_prefetch=0,
            grid=(npar, nblk),
            in_specs=[
                pl.BlockSpec((rows_blk, _LANES),
                             lambda i, j: (i * nblk + j, 0)),
                pl.BlockSpec(memory_space=pltpu.MemorySpace.SMEM),
            ],
            out_specs=[
                pl.BlockSpec((rows_blk // _LANES, _LANES),
                             lambda i, j: (i * nblk + j, 0))
                for _ in range(3)
            ],
        ),
        compiler_params=pltpu.CompilerParams(
            dimension_semantics=("parallel", "arbitrary")),
    )(x2, p2_vec)
    u1 = u1_2d.reshape(-1)
    u2 = u2_2d.reshape(-1)
    ol_all = ol_2d.reshape(-1)

    hoo1 = 0.5 * oo1
    p_vec = jnp.stack([hoo1, 1.0 - hoo1, 0.5 * k_oo, 0.5 * a_oo, sig_yvm,
                       exp_yrm, thr, obs_std]).astype(jnp.float32)

    _kernel_fn = functools.partial(_rnn_kernel, batch=batch, seq_len=seq,
                                   time_lag=time_lag)

    out = pl.pallas_call(
        _kernel_fn,
        out_shape=jax.ShapeDtypeStruct((_LANES, batch), jnp.float32),
        grid_spec=pltpu.PrefetchScalarGridSpec(
            num_scalar_prefetch=0,
            grid=(1,),
            in_specs=[
                pl.BlockSpec(memory_space=pltpu.MemorySpace.SMEM),
                pl.BlockSpec(memory_space=pltpu.MemorySpace.SMEM),
                pl.BlockSpec(memory_space=pltpu.MemorySpace.SMEM),
                pl.BlockSpec(memory_space=pltpu.MemorySpace.SMEM),
            ],
            out_specs=pl.BlockSpec((_LANES, batch), lambda i: (0, 0)),
            scratch_shapes=[pltpu.VMEM((1, _LANES), jnp.float32),
                            pltpu.VMEM((_GROUP, _LANES), jnp.float32)],
        ),
        compiler_params=pltpu.CompilerParams(
            dimension_semantics=("arbitrary",)),
    )(u1, u2, ol_all, p_vec)

    col = lambda j: out[j].reshape(batch, 1)
    h_n = col(_COL_H)
    obs_std_col = col(_COL_STD)
    h_nout = jnp.concatenate([h_n, obs_std_col], axis=1)
    return (h_n, col(_COL_C), col(_COL_L), col(_COL_LC), col(_COL_BP),
            col(_COL_IB), col(_COL_OO), col(_COL_OL), col(_COL_OLC),
            col(_COL_F), h_nout, obs_std_col, col(_COL_OV))


def kernel(x, y_obs, weight_r_yom, weight_r_ylm, weight_r_yfm, weight_r_yvm,
           bias_b0_yom, weight_b1_yom, bias_b0_ylm, weight_b2_ylm,
           bias_b0_yrm, p_mean, p_std):
    params = {
        'weight_r_yom': weight_r_yom,
        'weight_r_ylm': weight_r_ylm,
        'weight_r_yfm': weight_r_yfm,
        'weight_r_yvm': weight_r_yvm,
        'bias_b0_yom': bias_b0_yom,
        'weight_b1_yom': weight_b1_yom,
        'bias_b0_ylm': bias_b0_ylm,
        'weight_b2_ylm': weight_b2_ylm,
        'bias_b0_yrm': bias_b0_yrm,
    }
    return _forward(x, y_obs, params, p_mean, p_std,
                    time_lag=128, spin_len=128, train_len=4096)


# speedup vs baseline: 1.1002x; 1.0084x over previous
import functools

import jax
import jax.numpy as jnp
from jax import lax
from jax.experimental import pallas as pl
from jax.experimental.pallas import tpu as pltpu

_ML = 2.9086
_SL = 1.898
_SCALE_MR = 500.0
_INV_SCALE_MR = 1.0 / _SCALE_MR
_LANES = 128

_COL_H = 0
_COL_C = 1
_COL_L = 2
_COL_LC = 3
_COL_BP = 4
_COL_IB = 5
_COL_OO = 6
_COL_OL = 7
_COL_OLC = 8
_COL_F = 9
_COL_STD = 10
_COL_OV = 11

(_P_HOO1, _P_G1, _P_KOOH, _P_AOOH, _P_SIG, _P_EXP, _P_THR, _P_STD) = range(8)
_N_PARAMS = 8

_GROUP = 128


def _round_up(x, m):
    return (x + m - 1) // m * m


def _extract_kernel(x_ref, p2_ref, u1_ref, u2_ref, ol_ref):
    rows = x_ref.shape[0]
    ol1 = jnp.full((1, _LANES), p2_ref[0], dtype=jnp.float32)
    k_ol = jnp.full((1, _LANES), p2_ref[1], dtype=jnp.float32)
    a_ol = jnp.full((1, _LANES), p2_ref[2], dtype=jnp.float32)
    for g in range(rows // _LANES):
        t = x_ref[pl.ds(g * _LANES, _LANES), 0:8].T
        u2r = t[1:2, :]
        u1_ref[pl.ds(g, 1), :] = t[0:1, :]
        u2_ref[pl.ds(g, 1), :] = u2r
        ol_ref[pl.ds(g, 1), :] = ol1 * jax.nn.sigmoid(k_ol + u2r * a_ol)


def _rnn_kernel(u1_hbm, u2_hbm, ol_hbm, p_ref, out_ref, c_state, scr_ref,
                u1_ref, u2_ref, ol_ref, sems, *, batch, seq_len, time_lag):
    n = batch * seq_len
    n_groups = batch // _GROUP
    chunk_groups = min(8, n_groups)
    n_chunks = n_groups // chunk_groups
    chunk = n // n_chunks

    def _copies(c, slot):
        off = c * chunk
        return [
            pltpu.make_async_copy(ref_h.at[pl.ds(off, chunk)],
                                  ref_s.at[pl.ds(off, chunk)],
                                  sems.at[slot, j])
            for j, (ref_h, ref_s) in enumerate(
                ((u1_hbm, u1_ref), (u2_hbm, u2_ref), (ol_hbm, ol_ref)))
        ]

    shape = (1, _LANES)

    def splat(j):
        return jnp.full(shape, p_ref[j], dtype=jnp.float32)

    hoo1 = splat(_P_HOO1)
    g1 = splat(_P_G1)
    koo_h = splat(_P_KOOH)
    aoo_h = splat(_P_AOOH)
    sig = splat(_P_SIG)
    exp_yrm = splat(_P_EXP)
    thr = splat(_P_THR)
    obs_std = splat(_P_STD)
    lane = lax.broadcasted_iota(jnp.int32, shape, 1)
    _used = (_COL_H, _COL_C, _COL_L, _COL_LC, _COL_OO, _COL_OL,
             _COL_OLC, _COL_F, _COL_STD, _COL_OV)
    onehot = {j: (lane == j).astype(jnp.float32) for j in _used}

    def step(c0pair, u1, u2, ol):
        c0a, c0b = c0pair
        c0 = jnp.maximum(c0a, c0b)
        cpos = c0 > 0.0
        ta = jnp.tanh(koo_h + c0a * aoo_h)
        tb = jnp.tanh(koo_h + c0b * aoo_h)
        t = jnp.where(c0a >= c0b, ta, tb)
        r = pl.reciprocal(c0, approx=True)
        d = c0 - thr
        ad = jnp.abs(d)
        cm = c0 - ad
        e = u1 - sig * d
        q = ol * c0
        olc_c0 = jnp.where(cpos, jnp.minimum(q, u2), q)
        olc = jnp.where(cpos, jnp.minimum(ol, u2 * r), ol)
        hc = hoo1 * c0
        hcm = hoo1 * cm
        k1 = (g1 * c0 + e) - olc_c0
        k2 = g1 * cm + u1
        c1a = k1 - t * hc
        c1b = (k2 - t * hcm) - olc * cm
        return (c1a, c1b), t, olc, q, olc_c0

    out_ref[:, pl.ds(0, time_lag)] = jnp.zeros((_LANES, time_lag),
                                               out_ref.dtype)

    def row_body(g, k, cp0):
        row_off = (g * _GROUP + k) * seq_len
        cp = cp0
        for t in range(seq_len - 1):
            cp = step(cp, u1_ref[row_off + t], u2_ref[row_off + t],
                      ol_ref[row_off + t])[0]
        idx = row_off + seq_len - 1
        ol = ol_ref[idx]
        c = jnp.maximum(cp[0], cp[1])
        cp_new, t, olc, q, olc_c0 = step(cp, u1_ref[idx], u2_ref[idx], ol)
        a1 = hoo1 * t
        oo = hoo1 + a1
        f = (g1 - a1) - olc
        sgn = jnp.sign(c * _INV_SCALE_MR - exp_yrm)
        ov = jnp.minimum(sig * sgn, f)
        packed = ((oo * c) * onehot[_COL_H]
                  + c * onehot[_COL_C]
                  + q * onehot[_COL_L]
                  + olc_c0 * onehot[_COL_LC]
                  + oo * onehot[_COL_OO]
                  + ol * onehot[_COL_OL]
                  + olc * onehot[_COL_OLC]
                  + f * onehot[_COL_F]
                  + obs_std * onehot[_COL_STD]
                  + ov * onehot[_COL_OV])
        scr_ref[pl.ds(k, 1), :] = packed
        return cp_new

    def group_body(g, cp0):
        cp1 = lax.fori_loop(0, _GROUP,
                            lambda k, cp: row_body(g, k, cp), cp0, unroll=128)
        out_ref[:, pl.ds(g * _GROUP, _GROUP)] = scr_ref[...].T
        return cp1

    zero = jnp.zeros(shape, jnp.float32)
    g_first = time_lag // _GROUP
    for cpy in _copies(0, 0):
        cpy.start()
    if n_chunks > 1:
        for cpy in _copies(1, 1):
            cpy.start()
    cp = (zero, zero)
    for c in range(n_chunks):
        for cpy in _copies(c, c % 2):
            cpy.wait()
        cp = lax.fori_loop(max(c * chunk_groups, g_first),
                           (c + 1) * chunk_groups, group_body, cp)
        if c + 2 < n_chunks:
            for cpy in _copies(c + 2, c % 2):
                cpy.start()
    c_state[...] = jnp.maximum(cp[0], cp[1])


def _forward(x, y_obs, params, p_mean, p_std, *, time_lag, spin_len,
             train_len):
    batch, seq, _ = x.shape
    x = x.astype(jnp.float32)

    f32 = lambda v: jnp.asarray(v, jnp.float32).reshape(())
    w_r_yom = f32(params['weight_r_yom'])
    w_r_ylm = f32(params['weight_r_ylm'])
    w_r_yfm = f32(params['weight_r_yfm'])
    w_r_yvm = f32(params['weight_r_yvm'])
    b0_yom = f32(params['bias_b0_yom'])
    w_b1_yom = f32(params['weight_b1_yom'])
    b0_ylm = f32(params['bias_b0_ylm'])
    w_b2_ylm = f32(params['weight_b2_ylm'])
    b0_yrm = f32(params['bias_b0_yrm'])
    mo = f32(p_mean)
    so = f32(p_std)

    e_o, e_l, e_f = jnp.exp(w_r_yom), jnp.exp(w_r_ylm), jnp.exp(w_r_yfm)
    denom = e_o + e_l + e_f
    oo1 = e_o / denom
    ol1 = e_l / denom
    sig_yvm = jax.nn.sigmoid(w_r_yvm)
    exp_yrm = jnp.exp(b0_yrm)
    thr = exp_yrm * jnp.float32(_SCALE_MR)
    a_oo = w_b1_yom / so
    k_oo = b0_yom - mo * a_oo
    a_ol = w_b2_ylm / jnp.float32(_SL)
    k_ol = b0_ylm - jnp.float32(_ML) * a_ol
    obs_std = jnp.std(y_obs[spin_len:train_len].astype(jnp.float32), ddof=1)

    n = batch * seq
    x2 = x.reshape(n, _LANES)
    p2_vec = jnp.stack([ol1, k_ol, a_ol]).astype(jnp.float32)
    npar = 2
    nblk = max(1, n // (npar * 8192))
    rows_blk = n // (npar * nblk)
    u1_2d, u2_2d, ol_2d = pl.pallas_call(
        _extract_kernel,
        out_shape=[jax.ShapeDtypeStruct((n // _LANES, _LANES), jnp.float32)
                   for _ in range(3)],
        grid_spec=pltpu.PrefetchScalarGridSpec(
            num_scalar_prefetch=0,
            grid=(npar, nblk),
            in_specs=[
                pl.BlockSpec((rows_blk, _LANES),
                             lambda i, j: (i * nblk + j, 0)),
                pl.BlockSpec(memory_space=pltpu.MemorySpace.SMEM),
            ],
            out_specs=[
                pl.BlockSpec((rows_blk // _LANES, _LANES),
                             lambda i, j: (i * nblk + j, 0))
                for _ in range(3)
            ],
        ),
        compiler_params=pltpu.CompilerParams(
            dimension_semantics=("parallel", "arbitrary")),
    )(x2, p2_vec)
    u1 = u1_2d.reshape(-1)
    u2 = u2_2d.reshape(-1)
    ol_all = ol_2d.reshape(-1)

    hoo1 = 0.5 * oo1
    p_vec = jnp.stack([hoo1, 1.0 - hoo1, 0.5 * k_oo, 0.5 * a_oo, sig_yvm,
                       exp_yrm, thr, obs_std]).astype(jnp.float32)

    _kernel_fn = functools.partial(_rnn_kernel, batch=batch, seq_len=seq,
                                   time_lag=time_lag)

    out = pl.pallas_call(
        _kernel_fn,
        out_shape=jax.ShapeDtypeStruct((_LANES, batch), jnp.float32),
        grid_spec=pltpu.PrefetchScalarGridSpec(
            num_scalar_prefetch=0,
            grid=(1,),
            in_specs=[
                pl.BlockSpec(memory_space=pltpu.MemorySpace.HBM),
                pl.BlockSpec(memory_space=pltpu.MemorySpace.HBM),
                pl.BlockSpec(memory_space=pltpu.MemorySpace.HBM),
                pl.BlockSpec(memory_space=pltpu.MemorySpace.SMEM),
            ],
            out_specs=pl.BlockSpec((_LANES, batch), lambda i: (0, 0)),
            scratch_shapes=[pltpu.VMEM((1, _LANES), jnp.float32),
                            pltpu.VMEM((_GROUP, _LANES), jnp.float32),
                            pltpu.SMEM((batch * seq,), jnp.float32),
                            pltpu.SMEM((batch * seq,), jnp.float32),
                            pltpu.SMEM((batch * seq,), jnp.float32),
                            pltpu.SemaphoreType.DMA((2, 3))],
        ),
        compiler_params=pltpu.CompilerParams(
            dimension_semantics=("arbitrary",)),
    )(u1, u2, ol_all, p_vec)

    col = lambda j: out[j].reshape(batch, 1)
    h_n = col(_COL_H)
    obs_std_col = col(_COL_STD)
    h_nout = jnp.concatenate([h_n, obs_std_col], axis=1)
    return (h_n, col(_COL_C), col(_COL_L), col(_COL_LC), col(_COL_BP),
            col(_COL_IB), col(_COL_OO), col(_COL_OL), col(_COL_OLC),
            col(_COL_F), h_nout, obs_std_col, col(_COL_OV))


def kernel(x, y_obs, weight_r_yom, weight_r_ylm, weight_r_yfm, weight_r_yvm,
           bias_b0_yom, weight_b1_yom, bias_b0_ylm, weight_b2_ylm,
           bias_b0_yrm, p_mean, p_std):
    params = {
        'weight_r_yom': weight_r_yom,
        'weight_r_ylm': weight_r_ylm,
        'weight_r_yfm': weight_r_yfm,
        'weight_r_yvm': weight_r_yvm,
        'bias_b0_yom': bias_b0_yom,
        'weight_b1_yom': weight_b1_yom,
        'bias_b0_ylm': bias_b0_ylm,
        'weight_b2_ylm': weight_b2_ylm,
        'bias_b0_yrm': bias_b0_yrm,
    }
    return _forward(x, y_obs, params, p_mean, p_std,
                    time_lag=128, spin_len=128, train_len=4096)


# speedup vs baseline: 1.1028x; 1.0024x over previous
import functools

import jax
import jax.numpy as jnp
from jax import lax
from jax.experimental import pallas as pl
from jax.experimental.pallas import tpu as pltpu

_ML = 2.9086
_SL = 1.898
_SCALE_MR = 500.0
_INV_SCALE_MR = 1.0 / _SCALE_MR
_LANES = 128

_COL_H = 0
_COL_C = 1
_COL_L = 2
_COL_LC = 3
_COL_BP = 4
_COL_IB = 5
_COL_OO = 6
_COL_OL = 7
_COL_OLC = 8
_COL_F = 9
_COL_STD = 10
_COL_OV = 11

(_P_HOO1, _P_G1, _P_KOOH, _P_AOOH, _P_SIG, _P_EXP, _P_THR, _P_STD) = range(8)
_N_PARAMS = 8

_GROUP = 128


def _round_up(x, m):
    return (x + m - 1) // m * m


def _extract_kernel(x_ref, p2_ref, u1_ref, u2_ref, ol_ref):
    rows = x_ref.shape[0]
    ol1 = jnp.full((1, _LANES), p2_ref[0], dtype=jnp.float32)
    k_ol = jnp.full((1, _LANES), p2_ref[1], dtype=jnp.float32)
    a_ol = jnp.full((1, _LANES), p2_ref[2], dtype=jnp.float32)
    for g in range(rows // _LANES):
        t = x_ref[pl.ds(g * _LANES, _LANES), 0:8].T
        u2r = t[1:2, :]
        u1_ref[pl.ds(g, 1), :] = t[0:1, :]
        u2_ref[pl.ds(g, 1), :] = u2r
        ol_ref[pl.ds(g, 1), :] = ol1 * jax.nn.sigmoid(k_ol + u2r * a_ol)


def _rnn_kernel(u1_hbm, u2_hbm, ol_hbm, p_ref, out_ref, c_state, scr_ref,
                u1_ref, u2_ref, ol_ref, sems, *, batch, seq_len, time_lag):
    n = batch * seq_len
    n_groups = batch // _GROUP
    chunk_groups = min(8, n_groups)
    n_chunks = n_groups // chunk_groups
    chunk = n // n_chunks

    def _copies(c, slot):
        off = c * chunk
        return [
            pltpu.make_async_copy(ref_h.at[pl.ds(off, chunk)],
                                  ref_s.at[pl.ds(off, chunk)],
                                  sems.at[slot, j])
            for j, (ref_h, ref_s) in enumerate(
                ((u1_hbm, u1_ref), (u2_hbm, u2_ref), (ol_hbm, ol_ref)))
        ]

    shape = (1, _LANES)

    def splat(j):
        return jnp.full(shape, p_ref[j], dtype=jnp.float32)

    hoo1 = splat(_P_HOO1)
    g1 = splat(_P_G1)
    koo_h = splat(_P_KOOH)
    aoo_h = splat(_P_AOOH)
    sig = splat(_P_SIG)
    exp_yrm = splat(_P_EXP)
    thr = splat(_P_THR)
    obs_std = splat(_P_STD)
    lane = lax.broadcasted_iota(jnp.int32, shape, 1)
    _used = (_COL_H, _COL_C, _COL_L, _COL_LC, _COL_OO, _COL_OL,
             _COL_OLC, _COL_F, _COL_STD, _COL_OV)
    onehot = {j: (lane == j).astype(jnp.float32) for j in _used}

    def step(c0pair, u1, u2, ol):
        c0a, c0b = c0pair
        c0 = jnp.maximum(c0a, c0b)
        cpos = c0 > 0.0
        ta = jnp.tanh(koo_h + c0a * aoo_h)
        tb = jnp.tanh(koo_h + c0b * aoo_h)
        t = jnp.where(c0a >= c0b, ta, tb)
        r = pl.reciprocal(c0, approx=True)
        d = c0 - thr
        ad = jnp.abs(d)
        cm = c0 - ad
        e = u1 - sig * d
        q = ol * c0
        olc_c0 = jnp.where(cpos, jnp.minimum(q, u2), q)
        olc = jnp.where(cpos, jnp.minimum(ol, u2 * r), ol)
        hc = hoo1 * c0
        hcm = hoo1 * cm
        k1 = (g1 * c0 + e) - olc_c0
        k2 = g1 * cm + u1
        c1a = k1 - t * hc
        c1b = (k2 - t * hcm) - olc * cm
        return (c1a, c1b), t, olc, q, olc_c0

    out_ref[:, pl.ds(0, time_lag)] = jnp.zeros((_LANES, time_lag),
                                               out_ref.dtype)

    def row_body(g, k, cp0):
        row_off = (g * _GROUP + k) * seq_len
        cp = cp0
        for t in range(seq_len - 1):
            cp = step(cp, u1_ref[row_off + t], u2_ref[row_off + t],
                      ol_ref[row_off + t])[0]
        idx = row_off + seq_len - 1
        ol = ol_ref[idx]
        c = jnp.maximum(cp[0], cp[1])
        cp_new, t, olc, q, olc_c0 = step(cp, u1_ref[idx], u2_ref[idx], ol)
        a1 = hoo1 * t
        oo = hoo1 + a1
        f = (g1 - a1) - olc
        sgn = jnp.sign(c * _INV_SCALE_MR - exp_yrm)
        ov = jnp.minimum(sig * sgn, f)
        packed = ((oo * c) * onehot[_COL_H]
                  + c * onehot[_COL_C]
                  + q * onehot[_COL_L]
                  + olc_c0 * onehot[_COL_LC]
                  + oo * onehot[_COL_OO]
                  + ol * onehot[_COL_OL]
                  + olc * onehot[_COL_OLC]
                  + f * onehot[_COL_F]
                  + obs_std * onehot[_COL_STD]
                  + ov * onehot[_COL_OV])
        scr_ref[pl.ds(k, 1), :] = packed
        return cp_new

    def group_body(g, cp0):
        cp1 = lax.fori_loop(0, _GROUP,
                            lambda k, cp: row_body(g, k, cp), cp0, unroll=128)
        out_ref[:, pl.ds(g * _GROUP, _GROUP)] = scr_ref[...].T
        return cp1

    zero = jnp.zeros(shape, jnp.float32)
    g_first = time_lag // _GROUP
    for cpy in _copies(0, 0):
        cpy.start()
    if n_chunks > 1:
        for cpy in _copies(1, 1):
            cpy.start()
    cp = (zero, zero)
    for c in range(n_chunks):
        for cpy in _copies(c, c % 2):
            cpy.wait()
        cp = lax.fori_loop(max(c * chunk_groups, g_first),
                           (c + 1) * chunk_groups, group_body, cp)
        if c + 2 < n_chunks:
            for cpy in _copies(c + 2, c % 2):
                cpy.start()
    c_state[...] = jnp.maximum(cp[0], cp[1])


def _forward(x, y_obs, params, p_mean, p_std, *, time_lag, spin_len,
             train_len):
    batch, seq, _ = x.shape
    x = x.astype(jnp.float32)

    f32 = lambda v: jnp.asarray(v, jnp.float32).reshape(())
    w_r_yom = f32(params['weight_r_yom'])
    w_r_ylm = f32(params['weight_r_ylm'])
    w_r_yfm = f32(params['weight_r_yfm'])
    w_r_yvm = f32(params['weight_r_yvm'])
    b0_yom = f32(params['bias_b0_yom'])
    w_b1_yom = f32(params['weight_b1_yom'])
    b0_ylm = f32(params['bias_b0_ylm'])
    w_b2_ylm = f32(params['weight_b2_ylm'])
    b0_yrm = f32(params['bias_b0_yrm'])
    mo = f32(p_mean)
    so = f32(p_std)

    e_o, e_l, e_f = jnp.exp(w_r_yom), jnp.exp(w_r_ylm), jnp.exp(w_r_yfm)
    denom = e_o + e_l + e_f
    oo1 = e_o / denom
    ol1 = e_l / denom
    sig_yvm = jax.nn.sigmoid(w_r_yvm)
    exp_yrm = jnp.exp(b0_yrm)
    thr = exp_yrm * jnp.float32(_SCALE_MR)
    a_oo = w_b1_yom / so
    k_oo = b0_yom - mo * a_oo
    a_ol = w_b2_ylm / jnp.float32(_SL)
    k_ol = b0_ylm - jnp.float32(_ML) * a_ol
    obs_std = jnp.std(y_obs[spin_len:train_len].astype(jnp.float32), ddof=1)

    n = batch * seq
    x2 = x.reshape(n, _LANES)
    p2_vec = jnp.stack([ol1, k_ol, a_ol]).astype(jnp.float32)
    npar = 2
    nblk = max(1, n // (npar * 16384))
    rows_blk = n // (npar * nblk)
    u1_2d, u2_2d, ol_2d = pl.pallas_call(
        _extract_kernel,
        out_shape=[jax.ShapeDtypeStruct((n // _LANES, _LANES), jnp.float32)
                   for _ in range(3)],
        grid_spec=pltpu.PrefetchScalarGridSpec(
            num_scalar_prefetch=0,
            grid=(npar, nblk),
            in_specs=[
                pl.BlockSpec((rows_blk, _LANES),
                             lambda i, j: (i * nblk + j, 0)),
                pl.BlockSpec(memory_space=pltpu.MemorySpace.SMEM),
            ],
            out_specs=[
                pl.BlockSpec((rows_blk // _LANES, _LANES),
                             lambda i, j: (i * nblk + j, 0))
                for _ in range(3)
            ],
        ),
        compiler_params=pltpu.CompilerParams(
            dimension_semantics=("parallel", "arbitrary")),
    )(x2, p2_vec)
    u1 = u1_2d.reshape(-1)
    u2 = u2_2d.reshape(-1)
    ol_all = ol_2d.reshape(-1)

    hoo1 = 0.5 * oo1
    p_vec = jnp.stack([hoo1, 1.0 - hoo1, 0.5 * k_oo, 0.5 * a_oo, sig_yvm,
                       exp_yrm, thr, obs_std]).astype(jnp.float32)

    _kernel_fn = functools.partial(_rnn_kernel, batch=batch, seq_len=seq,
                                   time_lag=time_lag)

    out = pl.pallas_call(
        _kernel_fn,
        out_shape=jax.ShapeDtypeStruct((_LANES, batch), jnp.float32),
        grid_spec=pltpu.PrefetchScalarGridSpec(
            num_scalar_prefetch=0,
            grid=(1,),
            in_specs=[
                pl.BlockSpec(memory_space=pltpu.MemorySpace.HBM),
                pl.BlockSpec(memory_space=pltpu.MemorySpace.HBM),
                pl.BlockSpec(memory_space=pltpu.MemorySpace.HBM),
                pl.BlockSpec(memory_space=pltpu.MemorySpace.SMEM),
            ],
            out_specs=pl.BlockSpec((_LANES, batch), lambda i: (0, 0)),
            scratch_shapes=[pltpu.VMEM((1, _LANES), jnp.float32),
                            pltpu.VMEM((_GROUP, _LANES), jnp.float32),
                            pltpu.SMEM((batch * seq,), jnp.float32),
                            pltpu.SMEM((batch * seq,), jnp.float32),
                            pltpu.SMEM((batch * seq,), jnp.float32),
                            pltpu.SemaphoreType.DMA((2, 3))],
        ),
        compiler_params=pltpu.CompilerParams(
            dimension_semantics=("arbitrary",)),
    )(u1, u2, ol_all, p_vec)

    col = lambda j: out[j].reshape(batch, 1)
    h_n = col(_COL_H)
    obs_std_col = col(_COL_STD)
    h_nout = jnp.concatenate([h_n, obs_std_col], axis=1)
    return (h_n, col(_COL_C), col(_COL_L), col(_COL_LC), col(_COL_BP),
            col(_COL_IB), col(_COL_OO), col(_COL_OL), col(_COL_OLC),
            col(_COL_F), h_nout, obs_std_col, col(_COL_OV))


def kernel(x, y_obs, weight_r_yom, weight_r_ylm, weight_r_yfm, weight_r_yvm,
           bias_b0_yom, weight_b1_yom, bias_b0_ylm, weight_b2_ylm,
           bias_b0_yrm, p_mean, p_std):
    params = {
        'weight_r_yom': weight_r_yom,
        'weight_r_ylm': weight_r_ylm,
        'weight_r_yfm': weight_r_yfm,
        'weight_r_yvm': weight_r_yvm,
        'bias_b0_yom': bias_b0_yom,
        'weight_b1_yom': weight_b1_yom,
        'bias_b0_ylm': bias_b0_ylm,
        'weight_b2_ylm': weight_b2_ylm,
        'bias_b0_yrm': bias_b0_yrm,
    }
    return _forward(x, y_obs, params, p_mean, p_std,
                    time_lag=128, spin_len=128, train_len=4096)
